# Initial kernel scaffold; baseline (speedup 1.0000x reference)
import jax
import jax.numpy as jnp
from jax.experimental import pallas as pl


def kernel(x, edge_index, batch, atom_emb, W1, b1, gamma1, beta1, W2, b2, gamma2, beta2, gen_W1, gen_b1, gen_W2, gen_b2, W_out, b_out):
    raise NotImplementedError("write your pallas kernel here")


# speedup vs baseline: 6.6422x; 6.6422x over previous
import functools

import jax
import jax.numpy as jnp
from jax import lax
from jax.experimental import pallas as pl
from jax.experimental.pallas import tpu as pltpu
from jax.experimental.pallas import tpu_sc as plsc

N = 10000
E = 320000
H = 128
NG = 256
NPAD = 10240
EPAD = 327680
ER = EPAD // 128
C = 128
RB = 16
NC = 2
NS = 16

F32 = jnp.float32


def _mesh():
    return plsc.VectorSubcoreMesh(core_axis_name="c", subcore_axis_name="s",
                                  num_cores=NC, num_subcores=NS)


def _sc_enc_deg_body(idx2d, dst2d, emb, h0p, degout, ibuf, rbuf, abuf,
                     dbuf, ones_b, zbufd, deg_sh, sem):
    c = lax.axis_index("c")
    s = lax.axis_index("s")
    w = s * NC + c

    @pl.loop(0, 40)
    def _(i):
        zbufd[pl.ds(i * 16, 16)] = jnp.zeros((16,), F32)

    @pl.loop(0, 8)
    def _(i):
        ones_b[pl.ds(i * 16, 16)] = jnp.ones((16,), F32)

    pltpu.sync_copy(zbufd, deg_sh.at[pl.ds(s * 640, 640)])

    @pl.loop(0, 4)
    def _(k):
        chunk = w * 4 + k
        pltpu.sync_copy(idx2d.at[pl.ds(chunk * 16, 16)], ibuf)
        for col in range(9):
            pltpu.async_copy(emb.at[ibuf.at[col]], rbuf, sem).wait()
            if col == 0:
                @pl.loop(0, 80)
                def _(r):
                    for v in range(8):
                        abuf[r, pl.ds(v * 16, 16)] = rbuf[r, pl.ds(v * 16, 16)]
            else:
                @pl.loop(0, 80)
                def _(r):
                    for v in range(8):
                        abuf[r, pl.ds(v * 16, 16)] = (
                            abuf[r, pl.ds(v * 16, 16)]
                            + rbuf[r, pl.ds(v * 16, 16)])
        pltpu.sync_copy(abuf, h0p.at[pl.ds(chunk * 80, 80)])

    plsc.subcore_barrier()

    @pl.loop(0, 5)
    def _(b):
        rowbase = c * (ER // 2) + s * 80 + b * RB
        pltpu.sync_copy(dst2d.at[pl.ds(rowbase, RB)], dbuf)

        @pl.loop(0, RB)
        def _(j):
            pltpu.sync_copy(ones_b, deg_sh.at[dbuf.at[j]], add=True)

    plsc.subcore_barrier()
    pltpu.sync_copy(deg_sh.at[pl.ds(s * 640, 640)],
                    degout.at[pl.ds(c * NPAD + s * 640, 640)])


def _make_sc_enc_deg():
    return functools.partial(
        pl.kernel,
        out_type=[
            jax.ShapeDtypeStruct((NPAD, H), F32),
            jax.ShapeDtypeStruct((NC * NPAD,), F32),
        ],
        mesh=_mesh(),
        scratch_types=[
            pltpu.VMEM((16, 80), jnp.int32),
            pltpu.VMEM((80, H), F32),
            pltpu.VMEM((80, H), F32),
            pltpu.VMEM((RB, C), jnp.int32),
            pltpu.VMEM((C,), F32),
            pltpu.VMEM((640,), F32),
            pltpu.VMEM_SHARED((NPAD,), F32),
            pltpu.SemaphoreType.DMA,
        ],
    )(_sc_enc_deg_body)


def _make_sc_edge_pass(split):
    nblocks = 5 if split else 10

    def body(table, s2d, d2d, accout, sbuf, dbuf, rbuf, zbuf, acc_sh, sem):
        cc = lax.axis_index("c")
        s = lax.axis_index("s")

        @pl.loop(0, 128)
        def _(r):
            for v in range(8):
                zbuf[r, pl.ds(v * 16, 16)] = jnp.zeros((16,), F32)

        for k in range(5):
            pltpu.sync_copy(zbuf, acc_sh.at[pl.ds(s * 640 + k * 128, 128)])
        plsc.subcore_barrier()

        @pl.loop(0, nblocks)
        def _(b):
            if split:
                srow = cc * (ER // 2) + s * 80 + b * RB
                drow = srow
            else:
                srow = cc * ER + s * 160 + b * RB
                drow = s * 160 + b * RB
            pltpu.sync_copy(s2d.at[pl.ds(srow, RB)], sbuf)
            pltpu.sync_copy(d2d.at[pl.ds(drow, RB)], dbuf)

            @pl.loop(0, RB)
            def _(j):
                pltpu.async_copy(table.at[sbuf.at[j]], rbuf, sem).wait()
                pltpu.sync_copy(rbuf, acc_sh.at[dbuf.at[j]], add=True)

        plsc.subcore_barrier()
        for k in range(5):
            pltpu.sync_copy(acc_sh.at[pl.ds(s * 640 + k * 128, 128)],
                            accout.at[cc, pl.ds(s * 640 + k * 128, 128)])

    return functools.partial(
        pl.kernel,
        out_type=[jax.ShapeDtypeStruct((NC, NPAD, H), F32)],
        mesh=_mesh(),
        scratch_types=[
            pltpu.VMEM((RB, C), jnp.int32),
            pltpu.VMEM((RB, C), jnp.int32),
            pltpu.VMEM((C, H), F32),
            pltpu.VMEM((128, H), F32),
            pltpu.VMEM_SHARED((NPAD, H), F32),
            pltpu.SemaphoreType.DMA,
        ],
    )(body)


def _tc_k1(h0_ref, deg_ref, w1_ref, y1_ref, dinv_ref):
    deg = deg_ref[...]
    dinv = lax.rsqrt(jnp.maximum(deg, 1.0))
    dinv_ref[...] = dinv
    xw = jnp.dot(h0_ref[...], w1_ref[...], preferred_element_type=F32)
    y1_ref[...] = dinv * xw


def _tc_post(acc0_ref, acc1_ref, y_ref, dinv_ref, b_ref, g_ref, be_ref,
             w2_ref, y2_ref):
    dinv = dinv_ref[...]
    z = dinv * (acc0_ref[...] + acc1_ref[...] + y_ref[...]) + b_ref[...]
    mu = jnp.mean(z, axis=0, keepdims=True)
    zc = z - mu
    var = jnp.mean(zc * zc, axis=0, keepdims=True)
    h = jnp.maximum(zc * lax.rsqrt(var + 1e-5) * g_ref[...] + be_ref[...], 0.0)
    y2_ref[...] = dinv * jnp.dot(h, w2_ref[...], preferred_element_type=F32)


def _tc_post2(acc0_ref, acc1_ref, y_ref, dinv_ref, b_ref, g_ref, be_ref,
              h2_ref, tab_ref):
    dinv = dinv_ref[...]
    z = dinv * (acc0_ref[...] + acc1_ref[...] + y_ref[...]) + b_ref[...]
    mu = jnp.mean(z, axis=0, keepdims=True)
    zc = z - mu
    var = jnp.mean(zc * zc, axis=0, keepdims=True)
    h2 = jnp.maximum(zc * lax.rsqrt(var + 1e-5) * g_ref[...] + be_ref[...],
                     0.0)
    h2_ref[...] = h2
    t = h2 + 1e-7
    m = jnp.max(t, axis=0, keepdims=True)
    a = jnp.exp(t - m)
    tab_ref[0:N, :] = a
    tab_ref[N:2 * N, :] = a * t


def _tc_k4(h2_ref, den_ref, num_ref, gw1_ref, gb1_ref, gw2_ref, gb2_ref,
           wo_ref, bo_ref, batch_ref, out_ref):
    aggr = num_ref[...] / (den_ref[...] + 1e-30)
    h3 = h2_ref[...] + aggr
    r = jnp.maximum(
        jnp.dot(h3, gw1_ref[...], preferred_element_type=F32) + gb1_ref[...],
        0.0)
    h4 = jnp.dot(r, gw2_ref[...], preferred_element_type=F32) + gb2_ref[...]
    gids = lax.broadcasted_iota(jnp.int32, (NG, N), 0)
    oh = (gids == batch_ref[...]).astype(F32)
    sums = jnp.dot(oh, h4, preferred_element_type=F32)
    cnt = jnp.sum(oh, axis=1, keepdims=True)
    pooled = sums / jnp.maximum(cnt, 1.0)
    out_ref[...] = (jnp.dot(pooled, wo_ref[...], preferred_element_type=F32)
                    + bo_ref[...])


def _pc(body, out_shapes):
    return pl.pallas_call(body, out_shape=out_shapes)


def kernel(x, edge_index, batch, atom_emb, W1, b1, gamma1, beta1, W2, b2,
           gamma2, beta2, gen_W1, gen_b1, gen_W2, gen_b2, W_out, b_out):
    x = x.astype(jnp.int32)
    src = edge_index[0].astype(jnp.int32)
    dst = edge_index[1].astype(jnp.int32)
    batch = batch.astype(jnp.int32)

    xi = x + 119 * jnp.arange(9, dtype=jnp.int32)[None, :]
    xi = jnp.pad(xi, ((0, NPAD - N), (0, 0)))
    idx2d = (xi.reshape(NPAD // 80, 80, 9)
             .transpose(0, 2, 1))
    idx2d = jnp.pad(idx2d, ((0, 0), (0, 7), (0, 0)))
    idx2d = idx2d.reshape(NPAD // 80 * 16, 80)
    emb_tab = atom_emb.reshape(9 * 119, H)

    src_p = jnp.concatenate([src, jnp.zeros((EPAD - E,), jnp.int32)])
    dst_p = jnp.concatenate(
        [dst, jnp.full((EPAD - E,), N, jnp.int32)])
    src2d = src_p.reshape(ER, C)
    dst2d = dst_p.reshape(ER, C)
    srcg = jnp.concatenate([src_p, src_p + N]).reshape(2 * ER, C)

    h0p, degp = _make_sc_enc_deg()(idx2d, dst2d, emb_tab)
    h0 = h0p[:N]
    degp = degp.reshape(NC, NPAD)
    deg = 1.0 + degp[0, :N] + degp[1, :N]

    y1, dinv = _pc(_tc_k1, [jax.ShapeDtypeStruct((N, H), F32),
                            jax.ShapeDtypeStruct((N, 1), F32)])(
        h0, deg.reshape(N, 1), W1)

    edge_pass_split = _make_sc_edge_pass(True)
    edge_pass_full = _make_sc_edge_pass(False)

    (acc1,) = edge_pass_split(y1, src2d, dst2d)
    (y2,) = _pc(_tc_post, [jax.ShapeDtypeStruct((N, H), F32)])(
        acc1[0, :N], acc1[1, :N], y1, dinv, b1.reshape(1, H),
        gamma1.reshape(1, H), beta1.reshape(1, H), W2)

    (acc2,) = edge_pass_split(y2, src2d, dst2d)
    h2, tab = _pc(_tc_post2, [jax.ShapeDtypeStruct((N, H), F32),
                              jax.ShapeDtypeStruct((2 * N, H), F32)])(
        acc2[0, :N], acc2[1, :N], y2, dinv, b2.reshape(1, H),
        gamma2.reshape(1, H), beta2.reshape(1, H))

    (gen,) = edge_pass_full(tab, srcg, dst2d)
    den, num = gen[0, :N], gen[1, :N]

    (out,) = _pc(_tc_k4, [jax.ShapeDtypeStruct((NG, H), F32)])(
        h2, den, num, gen_W1, gen_b1.reshape(1, 2 * H), gen_W2,
        gen_b2.reshape(1, H), W_out, b_out.reshape(1, H),
        batch.reshape(1, N))
    return out


# speedup vs baseline: 7.4285x; 1.1184x over previous
import functools

import jax
import jax.numpy as jnp
from jax import lax
from jax.experimental import pallas as pl
from jax.experimental.pallas import tpu as pltpu
from jax.experimental.pallas import tpu_sc as plsc

N = 10000
E = 320000
H = 128
NG = 256
NPAD = 10240
EPAD = 327680
ER = EPAD // 128
C = 128
RB = 16
NC = 2
NS = 16

F32 = jnp.float32


def _mesh():
    return plsc.VectorSubcoreMesh(core_axis_name="c", subcore_axis_name="s",
                                  num_cores=NC, num_subcores=NS)


def _sc_enc_deg_body(idx2d, dst2d, emb, h0p, degout, ibuf, rbuf, abuf,
                     dbuf, ones_b, zbufd, deg_sh, sem):
    c = lax.axis_index("c")
    s = lax.axis_index("s")
    w = s * NC + c

    @pl.loop(0, 40)
    def _(i):
        zbufd[pl.ds(i * 16, 16)] = jnp.zeros((16,), F32)

    @pl.loop(0, 8)
    def _(i):
        ones_b[pl.ds(i * 16, 16)] = jnp.ones((16,), F32)

    pltpu.sync_copy(zbufd, deg_sh.at[pl.ds(s * 640, 640)])

    @pl.loop(0, 4)
    def _(k):
        chunk = w * 4 + k
        pltpu.sync_copy(idx2d.at[pl.ds(chunk * 16, 16)], ibuf)
        for col in range(9):
            pltpu.async_copy(emb.at[ibuf.at[col]], rbuf, sem).wait()
            if col == 0:
                @pl.loop(0, 80)
                def _(r):
                    for v in range(8):
                        abuf[r, pl.ds(v * 16, 16)] = rbuf[r, pl.ds(v * 16, 16)]
            else:
                @pl.loop(0, 80)
                def _(r):
                    for v in range(8):
                        abuf[r, pl.ds(v * 16, 16)] = (
                            abuf[r, pl.ds(v * 16, 16)]
                            + rbuf[r, pl.ds(v * 16, 16)])
        pltpu.sync_copy(abuf, h0p.at[pl.ds(chunk * 80, 80)])

    plsc.subcore_barrier()

    @pl.loop(0, 5)
    def _(b):
        rowbase = c * (ER // 2) + s * 80 + b * RB
        pltpu.sync_copy(dst2d.at[pl.ds(rowbase, RB)], dbuf)

        @pl.loop(0, RB)
        def _(j):
            pltpu.sync_copy(ones_b, deg_sh.at[dbuf.at[j]], add=True)

    plsc.subcore_barrier()
    pltpu.sync_copy(deg_sh.at[pl.ds(s * 640, 640)],
                    degout.at[pl.ds(c * NPAD + s * 640, 640)])


def _make_sc_enc_deg():
    return functools.partial(
        pl.kernel,
        out_type=[
            jax.ShapeDtypeStruct((NPAD, H), F32),
            jax.ShapeDtypeStruct((NC * NPAD,), F32),
        ],
        mesh=_mesh(),
        scratch_types=[
            pltpu.VMEM((16, 80), jnp.int32),
            pltpu.VMEM((80, H), F32),
            pltpu.VMEM((80, H), F32),
            pltpu.VMEM((RB, C), jnp.int32),
            pltpu.VMEM((C,), F32),
            pltpu.VMEM((640,), F32),
            pltpu.VMEM_SHARED((NPAD,), F32),
            pltpu.SemaphoreType.DMA,
        ],
    )(_sc_enc_deg_body)


def _make_sc_edge_pass(split):
    nblocks = 5 if split else 10
    NSLOT = 2

    def body(table, s2d, d2d, accout, sbuf, dbuf, rb0, rb1, acc_sh, gs0, gs1):
        rbufs = (rb0, rb1)
        gsems = (gs0, gs1)
        cc = lax.axis_index("c")
        s = lax.axis_index("s")

        @pl.loop(0, 128)
        def _(r):
            for v in range(8):
                rb0[r, pl.ds(v * 16, 16)] = jnp.zeros((16,), F32)

        for k in range(5):
            pltpu.sync_copy(rb0, acc_sh.at[pl.ds(s * 640 + k * 128, 128)])
        plsc.subcore_barrier()

        @pl.loop(0, nblocks)
        def _(b):
            if split:
                srow = cc * (ER // 2) + s * 80 + b * RB
                drow = srow
            else:
                srow = cc * ER + s * 160 + b * RB
                drow = s * 160 + b * RB
            pltpu.sync_copy(s2d.at[pl.ds(srow, RB)], sbuf)
            pltpu.sync_copy(d2d.at[pl.ds(drow, RB)], dbuf)

            gd = [None] * RB
            for j in range(RB):
                slot = j % NSLOT
                gd[j] = pltpu.async_copy(table.at[sbuf.at[j]], rbufs[slot],
                                         gsems[slot])
                if j >= NSLOT - 1:
                    k = j - (NSLOT - 1)
                    gd[k].wait()
                    pltpu.sync_copy(rbufs[k % NSLOT], acc_sh.at[dbuf.at[k]],
                                    add=True)
            for k in range(RB - NSLOT + 1, RB):
                gd[k].wait()
                pltpu.sync_copy(rbufs[k % NSLOT], acc_sh.at[dbuf.at[k]],
                                add=True)

        plsc.subcore_barrier()
        for k in range(5):
            pltpu.sync_copy(acc_sh.at[pl.ds(s * 640 + k * 128, 128)],
                            accout.at[cc, pl.ds(s * 640 + k * 128, 128)])

    return functools.partial(
        pl.kernel,
        out_type=[jax.ShapeDtypeStruct((NC, NPAD, H), F32)],
        mesh=_mesh(),
        scratch_types=[
            pltpu.VMEM((RB, C), jnp.int32),
            pltpu.VMEM((RB, C), jnp.int32),
            pltpu.VMEM((C, H), F32),
            pltpu.VMEM((C, H), F32),
            pltpu.VMEM_SHARED((NPAD, H), F32),
            pltpu.SemaphoreType.DMA,
            pltpu.SemaphoreType.DMA,
        ],
    )(body)


def _tc_k1(h0_ref, deg_ref, w1_ref, y1_ref, dinv_ref):
    deg = deg_ref[...]
    dinv = lax.rsqrt(jnp.maximum(deg, 1.0))
    dinv_ref[...] = dinv
    xw = jnp.dot(h0_ref[...], w1_ref[...], preferred_element_type=F32)
    y1_ref[...] = dinv * xw


def _tc_post(acc0_ref, acc1_ref, y_ref, dinv_ref, b_ref, g_ref, be_ref,
             w2_ref, y2_ref):
    dinv = dinv_ref[...]
    z = dinv * (acc0_ref[...] + acc1_ref[...] + y_ref[...]) + b_ref[...]
    mu = jnp.mean(z, axis=0, keepdims=True)
    zc = z - mu
    var = jnp.mean(zc * zc, axis=0, keepdims=True)
    h = jnp.maximum(zc * lax.rsqrt(var + 1e-5) * g_ref[...] + be_ref[...], 0.0)
    y2_ref[...] = dinv * jnp.dot(h, w2_ref[...], preferred_element_type=F32)


def _tc_post2(acc0_ref, acc1_ref, y_ref, dinv_ref, b_ref, g_ref, be_ref,
              h2_ref, tab_ref):
    dinv = dinv_ref[...]
    z = dinv * (acc0_ref[...] + acc1_ref[...] + y_ref[...]) + b_ref[...]
    mu = jnp.mean(z, axis=0, keepdims=True)
    zc = z - mu
    var = jnp.mean(zc * zc, axis=0, keepdims=True)
    h2 = jnp.maximum(zc * lax.rsqrt(var + 1e-5) * g_ref[...] + be_ref[...],
                     0.0)
    h2_ref[...] = h2
    t = h2 + 1e-7
    m = jnp.max(t, axis=0, keepdims=True)
    a = jnp.exp(t - m)
    tab_ref[0:N, :] = a
    tab_ref[N:2 * N, :] = a * t


def _tc_k4(h2_ref, den_ref, num_ref, gw1_ref, gb1_ref, gw2_ref, gb2_ref,
           wo_ref, bo_ref, batch_ref, out_ref):
    aggr = num_ref[...] / (den_ref[...] + 1e-30)
    h3 = h2_ref[...] + aggr
    r = jnp.maximum(
        jnp.dot(h3, gw1_ref[...], preferred_element_type=F32) + gb1_ref[...],
        0.0)
    h4 = jnp.dot(r, gw2_ref[...], preferred_element_type=F32) + gb2_ref[...]
    gids = lax.broadcasted_iota(jnp.int32, (NG, N), 0)
    oh = (gids == batch_ref[...]).astype(F32)
    sums = jnp.dot(oh, h4, preferred_element_type=F32)
    cnt = jnp.sum(oh, axis=1, keepdims=True)
    pooled = sums / jnp.maximum(cnt, 1.0)
    out_ref[...] = (jnp.dot(pooled, wo_ref[...], preferred_element_type=F32)
                    + bo_ref[...])


def _pc(body, out_shapes):
    return pl.pallas_call(body, out_shape=out_shapes)


def kernel(x, edge_index, batch, atom_emb, W1, b1, gamma1, beta1, W2, b2,
           gamma2, beta2, gen_W1, gen_b1, gen_W2, gen_b2, W_out, b_out):
    x = x.astype(jnp.int32)
    src = edge_index[0].astype(jnp.int32)
    dst = edge_index[1].astype(jnp.int32)
    batch = batch.astype(jnp.int32)

    xi = x + 119 * jnp.arange(9, dtype=jnp.int32)[None, :]
    xi = jnp.pad(xi, ((0, NPAD - N), (0, 0)))
    idx2d = (xi.reshape(NPAD // 80, 80, 9)
             .transpose(0, 2, 1))
    idx2d = jnp.pad(idx2d, ((0, 0), (0, 7), (0, 0)))
    idx2d = idx2d.reshape(NPAD // 80 * 16, 80)
    emb_tab = atom_emb.reshape(9 * 119, H)

    src_p = jnp.concatenate([src, jnp.zeros((EPAD - E,), jnp.int32)])
    dst_p = jnp.concatenate(
        [dst, jnp.full((EPAD - E,), N, jnp.int32)])
    src2d = src_p.reshape(ER, C)
    dst2d = dst_p.reshape(ER, C)
    srcg = jnp.concatenate([src_p, src_p + N]).reshape(2 * ER, C)

    h0p, degp = _make_sc_enc_deg()(idx2d, dst2d, emb_tab)
    h0 = h0p[:N]
    degp = degp.reshape(NC, NPAD)
    deg = 1.0 + degp[0, :N] + degp[1, :N]

    y1, dinv = _pc(_tc_k1, [jax.ShapeDtypeStruct((N, H), F32),
                            jax.ShapeDtypeStruct((N, 1), F32)])(
        h0, deg.reshape(N, 1), W1)

    edge_pass_split = _make_sc_edge_pass(True)
    edge_pass_full = _make_sc_edge_pass(False)

    (acc1,) = edge_pass_split(y1, src2d, dst2d)
    (y2,) = _pc(_tc_post, [jax.ShapeDtypeStruct((N, H), F32)])(
        acc1[0, :N], acc1[1, :N], y1, dinv, b1.reshape(1, H),
        gamma1.reshape(1, H), beta1.reshape(1, H), W2)

    (acc2,) = edge_pass_split(y2, src2d, dst2d)
    h2, tab = _pc(_tc_post2, [jax.ShapeDtypeStruct((N, H), F32),
                              jax.ShapeDtypeStruct((2 * N, H), F32)])(
        acc2[0, :N], acc2[1, :N], y2, dinv, b2.reshape(1, H),
        gamma2.reshape(1, H), beta2.reshape(1, H))

    (gen,) = edge_pass_full(tab, srcg, dst2d)
    den, num = gen[0, :N], gen[1, :N]

    (out,) = _pc(_tc_k4, [jax.ShapeDtypeStruct((NG, H), F32)])(
        h2, den, num, gen_W1, gen_b1.reshape(1, 2 * H), gen_W2,
        gen_b2.reshape(1, H), W_out, b_out.reshape(1, H),
        batch.reshape(1, N))
    return out


# speedup vs baseline: 7.5503x; 1.0164x over previous
import functools

import jax
import jax.numpy as jnp
from jax import lax
from jax.experimental import pallas as pl
from jax.experimental.pallas import tpu as pltpu
from jax.experimental.pallas import tpu_sc as plsc

N = 10000
E = 320000
H = 128
NG = 256
NPAD = 10240
EPAD = 327680
ER = EPAD // 128
C = 128
RB = 16
NC = 2
NS = 16

F32 = jnp.float32


def _mesh():
    return plsc.VectorSubcoreMesh(core_axis_name="c", subcore_axis_name="s",
                                  num_cores=NC, num_subcores=NS)


def _sc_enc_deg_body(idx2d, dst2d, emb, h0p, degout, ibuf, rbuf, rbuf2, abuf,
                     dbuf, ones_b, zbufd, deg_sh, sem, sem2):
    c = lax.axis_index("c")
    s = lax.axis_index("s")
    w = s * NC + c

    @pl.loop(0, 40)
    def _(i):
        zbufd[pl.ds(i * 16, 16)] = jnp.zeros((16,), F32)

    @pl.loop(0, 8)
    def _(i):
        ones_b[pl.ds(i * 16, 16)] = jnp.ones((16,), F32)

    pltpu.sync_copy(zbufd, deg_sh.at[pl.ds(s * 640, 640)])

    @pl.loop(0, 4)
    def _(k):
        chunk = w * 4 + k
        pltpu.sync_copy(idx2d.at[pl.ds(chunk * 16, 16)], ibuf)
        rbufs = (rbuf, rbuf2)
        gsems = (sem, sem2)
        gd = [None] * 9
        gd[0] = pltpu.async_copy(emb.at[ibuf.at[0]], rbufs[0], gsems[0])
        for col in range(9):
            if col + 1 < 9:
                gd[col + 1] = pltpu.async_copy(emb.at[ibuf.at[col + 1]],
                                               rbufs[(col + 1) % 2],
                                               gsems[(col + 1) % 2])
            gd[col].wait()
            rb = rbufs[col % 2]
            if col == 0:
                @pl.loop(0, 80)
                def _(r):
                    for v in range(8):
                        abuf[r, pl.ds(v * 16, 16)] = rb[r, pl.ds(v * 16, 16)]
            else:
                @pl.loop(0, 80)
                def _(r):
                    for v in range(8):
                        abuf[r, pl.ds(v * 16, 16)] = (
                            abuf[r, pl.ds(v * 16, 16)]
                            + rb[r, pl.ds(v * 16, 16)])
        pltpu.sync_copy(abuf, h0p.at[pl.ds(chunk * 80, 80)])

    plsc.subcore_barrier()

    @pl.loop(0, 5)
    def _(b):
        rowbase = c * (ER // 2) + s * 80 + b * RB
        pltpu.sync_copy(dst2d.at[pl.ds(rowbase, RB)], dbuf)

        sd = [None] * RB
        for j in range(RB):
            sd[j] = pltpu.async_copy(ones_b, deg_sh.at[dbuf.at[j]], sem,
                                     add=True)
        for j in range(RB):
            sd[j].wait()

    plsc.subcore_barrier()
    pltpu.sync_copy(deg_sh.at[pl.ds(s * 640, 640)],
                    degout.at[pl.ds(c * NPAD + s * 640, 640)])


def _make_sc_enc_deg():
    return functools.partial(
        pl.kernel,
        out_type=[
            jax.ShapeDtypeStruct((NPAD, H), F32),
            jax.ShapeDtypeStruct((NC * NPAD,), F32),
        ],
        mesh=_mesh(),
        scratch_types=[
            pltpu.VMEM((16, 80), jnp.int32),
            pltpu.VMEM((80, H), F32),
            pltpu.VMEM((80, H), F32),
            pltpu.VMEM((80, H), F32),
            pltpu.VMEM((RB, C), jnp.int32),
            pltpu.VMEM((C,), F32),
            pltpu.VMEM((640,), F32),
            pltpu.VMEM_SHARED((NPAD,), F32),
            pltpu.SemaphoreType.DMA,
            pltpu.SemaphoreType.DMA,
        ],
    )(_sc_enc_deg_body)


def _make_sc_edge_pass(split):
    nblocks = 5 if split else 10
    NSLOT = 2

    def body(table, s2d, d2d, accout, sbuf, dbuf, rb0, rb1, acc_sh,
             gs0, gs1, ss0, ss1):
        rbufs = (rb0, rb1)
        gsems = (gs0, gs1)
        ssems = (ss0, ss1)
        cc = lax.axis_index("c")
        s = lax.axis_index("s")

        @pl.loop(0, 128)
        def _(r):
            for v in range(8):
                rb0[r, pl.ds(v * 16, 16)] = jnp.zeros((16,), F32)

        for k in range(5):
            pltpu.sync_copy(rb0, acc_sh.at[pl.ds(s * 640 + k * 128, 128)])
        plsc.subcore_barrier()

        @pl.loop(0, nblocks)
        def _(b):
            if split:
                srow = cc * (ER // 2) + s * 80 + b * RB
                drow = srow
            else:
                srow = cc * ER + s * 160 + b * RB
                drow = s * 160 + b * RB
            pltpu.sync_copy(s2d.at[pl.ds(srow, RB)], sbuf)
            pltpu.sync_copy(d2d.at[pl.ds(drow, RB)], dbuf)

            gd = [None] * RB
            sd = [None] * RB
            for j in range(RB):
                slot = j % NSLOT
                if j >= NSLOT:
                    sd[j - NSLOT].wait()
                gd[j] = pltpu.async_copy(table.at[sbuf.at[j]], rbufs[slot],
                                         gsems[slot])
                if j >= 1:
                    k = j - 1
                    gd[k].wait()
                    sd[k] = pltpu.async_copy(rbufs[k % NSLOT],
                                             acc_sh.at[dbuf.at[k]],
                                             ssems[k % NSLOT], add=True)
            gd[RB - 1].wait()
            sd[RB - 1] = pltpu.async_copy(rbufs[(RB - 1) % NSLOT],
                                          acc_sh.at[dbuf.at[RB - 1]],
                                          ssems[(RB - 1) % NSLOT], add=True)
            sd[RB - 2].wait()
            sd[RB - 1].wait()

        plsc.subcore_barrier()
        for k in range(5):
            pltpu.sync_copy(acc_sh.at[pl.ds(s * 640 + k * 128, 128)],
                            accout.at[cc, pl.ds(s * 640 + k * 128, 128)])

    return functools.partial(
        pl.kernel,
        out_type=[jax.ShapeDtypeStruct((NC, NPAD, H), F32)],
        mesh=_mesh(),
        scratch_types=[
            pltpu.VMEM((RB, C), jnp.int32),
            pltpu.VMEM((RB, C), jnp.int32),
            pltpu.VMEM((C, H), F32),
            pltpu.VMEM((C, H), F32),
            pltpu.VMEM_SHARED((NPAD, H), F32),
            pltpu.SemaphoreType.DMA,
            pltpu.SemaphoreType.DMA,
            pltpu.SemaphoreType.DMA,
            pltpu.SemaphoreType.DMA,
        ],
    )(body)


def _tc_k1(h0_ref, deg_ref, w1_ref, y1_ref, dinv_ref):
    deg = deg_ref[...]
    dinv = lax.rsqrt(jnp.maximum(deg, 1.0))
    dinv_ref[...] = dinv
    xw = jnp.dot(h0_ref[...], w1_ref[...], preferred_element_type=F32)
    y1_ref[...] = dinv * xw


def _tc_post(acc0_ref, acc1_ref, y_ref, dinv_ref, b_ref, g_ref, be_ref,
             w2_ref, y2_ref):
    dinv = dinv_ref[...]
    z = dinv * (acc0_ref[...] + acc1_ref[...] + y_ref[...]) + b_ref[...]
    mu = jnp.mean(z, axis=0, keepdims=True)
    zc = z - mu
    var = jnp.mean(zc * zc, axis=0, keepdims=True)
    h = jnp.maximum(zc * lax.rsqrt(var + 1e-5) * g_ref[...] + be_ref[...], 0.0)
    y2_ref[...] = dinv * jnp.dot(h, w2_ref[...], preferred_element_type=F32)


def _tc_post2(acc0_ref, acc1_ref, y_ref, dinv_ref, b_ref, g_ref, be_ref,
              h2_ref, tab_ref):
    dinv = dinv_ref[...]
    z = dinv * (acc0_ref[...] + acc1_ref[...] + y_ref[...]) + b_ref[...]
    mu = jnp.mean(z, axis=0, keepdims=True)
    zc = z - mu
    var = jnp.mean(zc * zc, axis=0, keepdims=True)
    h2 = jnp.maximum(zc * lax.rsqrt(var + 1e-5) * g_ref[...] + be_ref[...],
                     0.0)
    h2_ref[...] = h2
    t = h2 + 1e-7
    m = jnp.max(t, axis=0, keepdims=True)
    a = jnp.exp(t - m)
    tab_ref[0:N, :] = a
    tab_ref[N:2 * N, :] = a * t


def _tc_k4(h2_ref, den_ref, num_ref, gw1_ref, gb1_ref, gw2_ref, gb2_ref,
           wo_ref, bo_ref, batch_ref, out_ref):
    aggr = num_ref[...] / (den_ref[...] + 1e-30)
    h3 = h2_ref[...] + aggr
    r = jnp.maximum(
        jnp.dot(h3, gw1_ref[...], preferred_element_type=F32) + gb1_ref[...],
        0.0)
    h4 = jnp.dot(r, gw2_ref[...], preferred_element_type=F32) + gb2_ref[...]
    gids = lax.broadcasted_iota(jnp.int32, (NG, N), 0)
    oh = (gids == batch_ref[...]).astype(F32)
    sums = jnp.dot(oh, h4, preferred_element_type=F32)
    cnt = jnp.sum(oh, axis=1, keepdims=True)
    pooled = sums / jnp.maximum(cnt, 1.0)
    out_ref[...] = (jnp.dot(pooled, wo_ref[...], preferred_element_type=F32)
                    + bo_ref[...])


def _pc(body, out_shapes):
    return pl.pallas_call(body, out_shape=out_shapes)


def kernel(x, edge_index, batch, atom_emb, W1, b1, gamma1, beta1, W2, b2,
           gamma2, beta2, gen_W1, gen_b1, gen_W2, gen_b2, W_out, b_out):
    x = x.astype(jnp.int32)
    src = edge_index[0].astype(jnp.int32)
    dst = edge_index[1].astype(jnp.int32)
    batch = batch.astype(jnp.int32)

    xi = x + 119 * jnp.arange(9, dtype=jnp.int32)[None, :]
    xi = jnp.pad(xi, ((0, NPAD - N), (0, 0)))
    idx2d = (xi.reshape(NPAD // 80, 80, 9)
             .transpose(0, 2, 1))
    idx2d = jnp.pad(idx2d, ((0, 0), (0, 7), (0, 0)))
    idx2d = idx2d.reshape(NPAD // 80 * 16, 80)
    emb_tab = atom_emb.reshape(9 * 119, H)

    src_p = jnp.concatenate([src, jnp.zeros((EPAD - E,), jnp.int32)])
    dst_p = jnp.concatenate(
        [dst, jnp.full((EPAD - E,), N, jnp.int32)])
    src2d = src_p.reshape(ER, C)
    dst2d = dst_p.reshape(ER, C)
    srcg = jnp.concatenate([src_p, src_p + N]).reshape(2 * ER, C)

    h0p, degp = _make_sc_enc_deg()(idx2d, dst2d, emb_tab)
    h0 = h0p[:N]
    degp = degp.reshape(NC, NPAD)
    deg = 1.0 + degp[0, :N] + degp[1, :N]

    y1, dinv = _pc(_tc_k1, [jax.ShapeDtypeStruct((N, H), F32),
                            jax.ShapeDtypeStruct((N, 1), F32)])(
        h0, deg.reshape(N, 1), W1)

    edge_pass_split = _make_sc_edge_pass(True)
    edge_pass_full = _make_sc_edge_pass(False)

    (acc1,) = edge_pass_split(y1, src2d, dst2d)
    (y2,) = _pc(_tc_post, [jax.ShapeDtypeStruct((N, H), F32)])(
        acc1[0, :N], acc1[1, :N], y1, dinv, b1.reshape(1, H),
        gamma1.reshape(1, H), beta1.reshape(1, H), W2)

    (acc2,) = edge_pass_split(y2, src2d, dst2d)
    h2, tab = _pc(_tc_post2, [jax.ShapeDtypeStruct((N, H), F32),
                              jax.ShapeDtypeStruct((2 * N, H), F32)])(
        acc2[0, :N], acc2[1, :N], y2, dinv, b2.reshape(1, H),
        gamma2.reshape(1, H), beta2.reshape(1, H))

    (gen,) = edge_pass_full(tab, srcg, dst2d)
    den, num = gen[0, :N], gen[1, :N]

    (out,) = _pc(_tc_k4, [jax.ShapeDtypeStruct((NG, H), F32)])(
        h2, den, num, gen_W1, gen_b1.reshape(1, 2 * H), gen_W2,
        gen_b2.reshape(1, H), W_out, b_out.reshape(1, H),
        batch.reshape(1, N))
    return out


# speedup vs baseline: 18.9803x; 2.5139x over previous
import functools

import jax
import jax.numpy as jnp
from jax import lax
from jax.experimental import pallas as pl
from jax.experimental.pallas import tpu as pltpu
from jax.experimental.pallas import tpu_sc as plsc

N = 10000
E = 320000
H = 128
NG = 256
NPAD = 10240
EPAD = 327680
ER = EPAD // 128
C = 128
RB = 16
NC = 2
NS = 16

F32 = jnp.float32


def _mesh():
    return plsc.VectorSubcoreMesh(core_axis_name="c", subcore_axis_name="s",
                                  num_cores=NC, num_subcores=NS)


def _sc_enc_deg_body(idx2d, dst2d, emb, h0p, degout, ibuf, rbuf, rbuf2, abuf,
                     dbuf, ones_b, zbufd, deg_sh, sem, sem2):
    c = lax.axis_index("c")
    s = lax.axis_index("s")
    w = s * NC + c

    @pl.loop(0, 40)
    def _(i):
        zbufd[pl.ds(i * 16, 16)] = jnp.zeros((16,), F32)

    @pl.loop(0, 8)
    def _(i):
        ones_b[pl.ds(i * 16, 16)] = jnp.ones((16,), F32)

    pltpu.sync_copy(zbufd, deg_sh.at[pl.ds(s * 640, 640)])

    @pl.loop(0, 4)
    def _(k):
        chunk = w * 4 + k
        pltpu.sync_copy(idx2d.at[pl.ds(chunk * 16, 16)], ibuf)
        rbufs = (rbuf, rbuf2)
        gsems = (sem, sem2)
        gd = [None] * 9
        gd[0] = pltpu.async_copy(emb.at[ibuf.at[0]], rbufs[0], gsems[0])
        for col in range(9):
            if col + 1 < 9:
                gd[col + 1] = pltpu.async_copy(emb.at[ibuf.at[col + 1]],
                                               rbufs[(col + 1) % 2],
                                               gsems[(col + 1) % 2])
            gd[col].wait()
            rb = rbufs[col % 2]
            if col == 0:
                @pl.loop(0, 80)
                def _(r):
                    for v in range(8):
                        abuf[r, pl.ds(v * 16, 16)] = rb[r, pl.ds(v * 16, 16)]
            else:
                @pl.loop(0, 80)
                def _(r):
                    for v in range(8):
                        abuf[r, pl.ds(v * 16, 16)] = (
                            abuf[r, pl.ds(v * 16, 16)]
                            + rb[r, pl.ds(v * 16, 16)])
        pltpu.sync_copy(abuf, h0p.at[pl.ds(chunk * 80, 80)])

    plsc.subcore_barrier()

    @pl.loop(0, 5)
    def _(b):
        rowbase = c * (ER // 2) + s * 80 + b * RB
        pltpu.sync_copy(dst2d.at[pl.ds(rowbase, RB)], dbuf)

        sd = [None] * RB
        for j in range(RB):
            sd[j] = pltpu.async_copy(ones_b, deg_sh.at[dbuf.at[j]], sem,
                                     add=True)
        for j in range(RB):
            sd[j].wait()

    plsc.subcore_barrier()
    pltpu.sync_copy(deg_sh.at[pl.ds(s * 640, 640)],
                    degout.at[pl.ds(c * NPAD + s * 640, 640)])


def _make_sc_enc_deg():
    return functools.partial(
        pl.kernel,
        out_type=[
            jax.ShapeDtypeStruct((NPAD, H), F32),
            jax.ShapeDtypeStruct((NC * NPAD,), F32),
        ],
        mesh=_mesh(),
        scratch_types=[
            pltpu.VMEM((16, 80), jnp.int32),
            pltpu.VMEM((80, H), F32),
            pltpu.VMEM((80, H), F32),
            pltpu.VMEM((80, H), F32),
            pltpu.VMEM((RB, C), jnp.int32),
            pltpu.VMEM((C,), F32),
            pltpu.VMEM((640,), F32),
            pltpu.VMEM_SHARED((NPAD,), F32),
            pltpu.SemaphoreType.DMA,
            pltpu.SemaphoreType.DMA,
        ],
    )(_sc_enc_deg_body)


def _make_sc_edge_pass(split):
    nblocks = 5 if split else 10
    NSLOT = 2

    def body(table, s2d, d2d, accout, sbuf, dbuf, rb0, rb1, acc_sh,
             gs0, gs1, ss0, ss1):
        rbufs = (rb0, rb1)
        gsems = (gs0, gs1)
        ssems = (ss0, ss1)
        cc = lax.axis_index("c")
        s = lax.axis_index("s")

        @pl.loop(0, 128)
        def _(r):
            for v in range(8):
                rb0[r, pl.ds(v * 16, 16)] = jnp.zeros((16,), F32)

        for k in range(5):
            pltpu.sync_copy(rb0, acc_sh.at[pl.ds(s * 640 + k * 128, 128)])
        plsc.subcore_barrier()

        @pl.loop(0, nblocks)
        def _(b):
            if split:
                srow = cc * (ER // 2) + s * 80 + b * RB
                drow = srow
            else:
                srow = cc * ER + s * 160 + b * RB
                drow = s * 160 + b * RB
            pltpu.sync_copy(s2d.at[pl.ds(srow, RB)], sbuf)
            pltpu.sync_copy(d2d.at[pl.ds(drow, RB)], dbuf)

            gd = [None] * RB
            sd = [None] * RB
            for j in range(RB):
                slot = j % NSLOT
                if j >= NSLOT:
                    sd[j - NSLOT].wait()
                gd[j] = pltpu.async_copy(table.at[sbuf.at[j]], rbufs[slot],
                                         gsems[slot])
                if j >= 1:
                    k = j - 1
                    gd[k].wait()
                    sd[k] = pltpu.async_copy(rbufs[k % NSLOT],
                                             acc_sh.at[dbuf.at[k]],
                                             ssems[k % NSLOT], add=True)
            gd[RB - 1].wait()
            sd[RB - 1] = pltpu.async_copy(rbufs[(RB - 1) % NSLOT],
                                          acc_sh.at[dbuf.at[RB - 1]],
                                          ssems[(RB - 1) % NSLOT], add=True)
            sd[RB - 2].wait()
            sd[RB - 1].wait()

        plsc.subcore_barrier()
        for k in range(5):
            pltpu.sync_copy(acc_sh.at[pl.ds(s * 640 + k * 128, 128)],
                            accout.at[cc, pl.ds(s * 640 + k * 128, 128)])

    return functools.partial(
        pl.kernel,
        out_type=[jax.ShapeDtypeStruct((NC, NPAD, H), F32)],
        mesh=_mesh(),
        scratch_types=[
            pltpu.VMEM((RB, C), jnp.int32),
            pltpu.VMEM((RB, C), jnp.int32),
            pltpu.VMEM((C, H), F32),
            pltpu.VMEM((C, H), F32),
            pltpu.VMEM_SHARED((NPAD, H), F32),
            pltpu.SemaphoreType.DMA,
            pltpu.SemaphoreType.DMA,
            pltpu.SemaphoreType.DMA,
            pltpu.SemaphoreType.DMA,
        ],
    )(body)


def _tc_k1(h0_ref, deg_ref, w1_ref, y1_ref, dinv_ref):
    deg = deg_ref[...]
    dinv = lax.rsqrt(jnp.maximum(deg, 1.0))
    dinv_ref[...] = dinv
    xw = jnp.dot(h0_ref[...], w1_ref[...], preferred_element_type=F32)
    y1_ref[...] = dinv * xw


def _tc_post(acc0_ref, acc1_ref, y_ref, dinv_ref, b_ref, g_ref, be_ref,
             w2_ref, y2_ref):
    dinv = dinv_ref[...]
    z = dinv * (acc0_ref[...] + acc1_ref[...] + y_ref[...]) + b_ref[...]
    mu = jnp.mean(z, axis=0, keepdims=True)
    zc = z - mu
    var = jnp.mean(zc * zc, axis=0, keepdims=True)
    h = jnp.maximum(zc * lax.rsqrt(var + 1e-5) * g_ref[...] + be_ref[...], 0.0)
    y2_ref[...] = dinv * jnp.dot(h, w2_ref[...], preferred_element_type=F32)


def _tc_post2(acc0_ref, acc1_ref, y_ref, dinv_ref, b_ref, g_ref, be_ref,
              h2_ref, tab_ref):
    dinv = dinv_ref[...]
    z = dinv * (acc0_ref[...] + acc1_ref[...] + y_ref[...]) + b_ref[...]
    mu = jnp.mean(z, axis=0, keepdims=True)
    zc = z - mu
    var = jnp.mean(zc * zc, axis=0, keepdims=True)
    h2 = jnp.maximum(zc * lax.rsqrt(var + 1e-5) * g_ref[...] + be_ref[...],
                     0.0)
    h2_ref[...] = h2
    t = h2 + 1e-7
    m = jnp.max(t, axis=0, keepdims=True)
    a = jnp.exp(t - m)
    tab_ref[0:N, :] = a
    tab_ref[N:2 * N, :] = a * t


def _tc_k4(h2_ref, den_ref, num_ref, gw1_ref, gb1_ref, gw2_ref, gb2_ref,
           wo_ref, bo_ref, batch_ref, out_ref):
    aggr = num_ref[...] / (den_ref[...] + 1e-30)
    h3 = h2_ref[...] + aggr
    r = jnp.maximum(
        jnp.dot(h3, gw1_ref[...], preferred_element_type=F32) + gb1_ref[...],
        0.0)
    h4 = jnp.dot(r, gw2_ref[...], preferred_element_type=F32) + gb2_ref[...]
    gids = lax.broadcasted_iota(jnp.int32, (NG, N), 0)
    oh = (gids == batch_ref[...]).astype(F32)
    sums = jnp.dot(oh, h4, preferred_element_type=F32)
    cnt = jnp.sum(oh, axis=1, keepdims=True)
    pooled = sums / jnp.maximum(cnt, 1.0)
    out_ref[...] = (jnp.dot(pooled, wo_ref[...], preferred_element_type=F32)
                    + bo_ref[...])


def _pc(body, out_shapes):
    return pl.pallas_call(body, out_shape=out_shapes)


def kernel(x, edge_index, batch, atom_emb, W1, b1, gamma1, beta1, W2, b2,
           gamma2, beta2, gen_W1, gen_b1, gen_W2, gen_b2, W_out, b_out):
    x = x.astype(jnp.int32)
    src = edge_index[0].astype(jnp.int32)
    dst = edge_index[1].astype(jnp.int32)
    batch = batch.astype(jnp.int32)

    xi = x + 119 * jnp.arange(9, dtype=jnp.int32)[None, :]
    xi = jnp.pad(xi, ((0, NPAD - N), (0, 0)))
    idx2d = (xi.reshape(NPAD // 80, 80, 9)
             .transpose(0, 2, 1))
    idx2d = jnp.pad(idx2d, ((0, 0), (0, 7), (0, 0)))
    idx2d = idx2d.reshape(NPAD // 80 * 16, 80)
    emb_tab = atom_emb.reshape(9 * 119, H)

    pad_ar = jnp.arange(EPAD - E, dtype=jnp.int32)
    src_p = jnp.concatenate([src, (pad_ar * 131) % N])
    dst_p = jnp.concatenate([dst, N + (pad_ar % (NPAD - N))])
    src2d = src_p.reshape(ER, C)
    dst2d = dst_p.reshape(ER, C)
    srcg = jnp.concatenate([src_p, src_p + N]).reshape(2 * ER, C)

    h0p, degp = _make_sc_enc_deg()(idx2d, dst2d, emb_tab)
    h0 = h0p[:N]
    degp = degp.reshape(NC, NPAD)
    deg = 1.0 + degp[0, :N] + degp[1, :N]

    y1, dinv = _pc(_tc_k1, [jax.ShapeDtypeStruct((N, H), F32),
                            jax.ShapeDtypeStruct((N, 1), F32)])(
        h0, deg.reshape(N, 1), W1)

    edge_pass_split = _make_sc_edge_pass(True)
    edge_pass_full = _make_sc_edge_pass(False)

    (acc1,) = edge_pass_split(y1, src2d, dst2d)
    (y2,) = _pc(_tc_post, [jax.ShapeDtypeStruct((N, H), F32)])(
        acc1[0, :N], acc1[1, :N], y1, dinv, b1.reshape(1, H),
        gamma1.reshape(1, H), beta1.reshape(1, H), W2)

    (acc2,) = edge_pass_split(y2, src2d, dst2d)
    h2, tab = _pc(_tc_post2, [jax.ShapeDtypeStruct((N, H), F32),
                              jax.ShapeDtypeStruct((2 * N, H), F32)])(
        acc2[0, :N], acc2[1, :N], y2, dinv, b2.reshape(1, H),
        gamma2.reshape(1, H), beta2.reshape(1, H))

    (gen,) = edge_pass_full(tab, srcg, dst2d)
    den, num = gen[0, :N], gen[1, :N]

    (out,) = _pc(_tc_k4, [jax.ShapeDtypeStruct((NG, H), F32)])(
        h2, den, num, gen_W1, gen_b1.reshape(1, 2 * H), gen_W2,
        gen_b2.reshape(1, H), W_out, b_out.reshape(1, H),
        batch.reshape(1, N))
    return out


# speedup vs baseline: 19.6968x; 1.0378x over previous
import functools

import jax
import jax.numpy as jnp
from jax import lax
from jax.experimental import pallas as pl
from jax.experimental.pallas import tpu as pltpu
from jax.experimental.pallas import tpu_sc as plsc

N = 10000
E = 320000
H = 128
NG = 256
NPAD = 10240
EPAD = 327680
ER = EPAD // 128
C = 128
RB = 16
NC = 2
NS = 16
DEGPAD = NPAD + (EPAD - E)

F32 = jnp.float32


def _mesh():
    return plsc.VectorSubcoreMesh(core_axis_name="c", subcore_axis_name="s",
                                  num_cores=NC, num_subcores=NS)


def _sc_enc_deg_body(idx2d, dst2d, emb, h0p, degout, ibuf, rbuf, rbuf2, abuf,
                     dbuf, ones_b, zbufd, deg_sh, sem, sem2):
    c = lax.axis_index("c")
    s = lax.axis_index("s")
    w = s * NC + c

    @pl.loop(0, 40)
    def _(i):
        zbufd[pl.ds(i * 16, 16)] = jnp.zeros((16,), F32)

    @pl.loop(0, 8)
    def _(i):
        ones_b[pl.ds(i * 16, 16)] = jnp.ones((16,), F32)

    pltpu.sync_copy(zbufd, deg_sh.at[pl.ds(s * 640, 640)])

    @pl.loop(0, 4)
    def _(k):
        chunk = w * 4 + k
        pltpu.sync_copy(idx2d.at[pl.ds(chunk * 16, 16)], ibuf)
        rbufs = (rbuf, rbuf2)
        gsems = (sem, sem2)
        gd = [None] * 9
        gd[0] = pltpu.async_copy(emb.at[ibuf.at[0]], rbufs[0], gsems[0])
        for col in range(9):
            if col + 1 < 9:
                gd[col + 1] = pltpu.async_copy(emb.at[ibuf.at[col + 1]],
                                               rbufs[(col + 1) % 2],
                                               gsems[(col + 1) % 2])
            gd[col].wait()
            rb = rbufs[col % 2]
            if col == 0:
                @pl.loop(0, 80)
                def _(r):
                    for v in range(8):
                        abuf[r, pl.ds(v * 16, 16)] = rb[r, pl.ds(v * 16, 16)]
            else:
                @pl.loop(0, 80)
                def _(r):
                    for v in range(8):
                        abuf[r, pl.ds(v * 16, 16)] = (
                            abuf[r, pl.ds(v * 16, 16)]
                            + rb[r, pl.ds(v * 16, 16)])
        pltpu.sync_copy(abuf, h0p.at[pl.ds(chunk * 80, 80)])

    plsc.subcore_barrier()

    @pl.loop(0, 5)
    def _(b):
        rowbase = c * (ER // 2) + s * 80 + b * RB
        pltpu.sync_copy(dst2d.at[pl.ds(rowbase, RB)], dbuf)

        sd = [None] * RB
        for j in range(RB):
            sd[j] = pltpu.async_copy(ones_b, deg_sh.at[dbuf.at[j]], sem,
                                     add=True)
        for j in range(RB):
            sd[j].wait()

    plsc.subcore_barrier()
    pltpu.sync_copy(deg_sh.at[pl.ds(s * 640, 640)],
                    degout.at[pl.ds(c * NPAD + s * 640, 640)])


def _make_sc_enc_deg():
    return functools.partial(
        pl.kernel,
        out_type=[
            jax.ShapeDtypeStruct((NPAD, H), F32),
            jax.ShapeDtypeStruct((NC * NPAD,), F32),
        ],
        mesh=_mesh(),
        scratch_types=[
            pltpu.VMEM((16, 80), jnp.int32),
            pltpu.VMEM((80, H), F32),
            pltpu.VMEM((80, H), F32),
            pltpu.VMEM((80, H), F32),
            pltpu.VMEM((RB, C), jnp.int32),
            pltpu.VMEM((C,), F32),
            pltpu.VMEM((640,), F32),
            pltpu.VMEM_SHARED((DEGPAD,), F32),
            pltpu.SemaphoreType.DMA,
            pltpu.SemaphoreType.DMA,
        ],
    )(_sc_enc_deg_body)


def _make_sc_edge_pass(split):
    nblocks = 5 if split else 10
    NSLOT = 2

    def body(table, s2d, d2d, accout, sbuf, dbuf, rb0, rb1, acc_sh,
             gs0, gs1, ss0, ss1):
        rbufs = (rb0, rb1)
        gsems = (gs0, gs1)
        ssems = (ss0, ss1)
        cc = lax.axis_index("c")
        s = lax.axis_index("s")

        @pl.loop(0, 128)
        def _(r):
            for v in range(8):
                rb0[r, pl.ds(v * 16, 16)] = jnp.zeros((16,), F32)

        for k in range(5):
            pltpu.sync_copy(rb0, acc_sh.at[pl.ds(s * 640 + k * 128, 128)])
        plsc.subcore_barrier()

        @pl.loop(0, nblocks)
        def _(b):
            if split:
                srow = cc * (ER // 2) + s * 80 + b * RB
                drow = srow
            else:
                srow = cc * ER + s * 160 + b * RB
                drow = s * 160 + b * RB
            pltpu.sync_copy(s2d.at[pl.ds(srow, RB)], sbuf)
            pltpu.sync_copy(d2d.at[pl.ds(drow, RB)], dbuf)

            gd = [None] * RB
            sd = [None] * RB
            for j in range(RB):
                slot = j % NSLOT
                if j >= NSLOT:
                    sd[j - NSLOT].wait()
                gd[j] = pltpu.async_copy(table.at[sbuf.at[j]], rbufs[slot],
                                         gsems[slot])
                if j >= 1:
                    k = j - 1
                    gd[k].wait()
                    sd[k] = pltpu.async_copy(rbufs[k % NSLOT],
                                             acc_sh.at[dbuf.at[k]],
                                             ssems[k % NSLOT], add=True)
            gd[RB - 1].wait()
            sd[RB - 1] = pltpu.async_copy(rbufs[(RB - 1) % NSLOT],
                                          acc_sh.at[dbuf.at[RB - 1]],
                                          ssems[(RB - 1) % NSLOT], add=True)
            sd[RB - 2].wait()
            sd[RB - 1].wait()

        plsc.subcore_barrier()
        for k in range(5):
            pltpu.sync_copy(acc_sh.at[pl.ds(s * 640 + k * 128, 128)],
                            accout.at[cc, pl.ds(s * 640 + k * 128, 128)])

    return functools.partial(
        pl.kernel,
        out_type=[jax.ShapeDtypeStruct((NC, NPAD, H), F32)],
        mesh=_mesh(),
        scratch_types=[
            pltpu.VMEM((RB, C), jnp.int32),
            pltpu.VMEM((RB, C), jnp.int32),
            pltpu.VMEM((C, H), F32),
            pltpu.VMEM((C, H), F32),
            pltpu.VMEM_SHARED((NPAD, H), F32),
            pltpu.SemaphoreType.DMA,
            pltpu.SemaphoreType.DMA,
            pltpu.SemaphoreType.DMA,
            pltpu.SemaphoreType.DMA,
        ],
    )(body)


def _tc_k1(h0_ref, deg_ref, w1_ref, y1_ref, dinv_ref):
    deg = deg_ref[...]
    dinv = lax.rsqrt(jnp.maximum(deg, 1.0))
    dinv_ref[...] = dinv
    xw = jnp.dot(h0_ref[...], w1_ref[...], preferred_element_type=F32)
    y1_ref[...] = dinv * xw


def _tc_post(acc0_ref, acc1_ref, y_ref, dinv_ref, b_ref, g_ref, be_ref,
             w2_ref, y2_ref):
    dinv = dinv_ref[...]
    z = dinv * (acc0_ref[...] + acc1_ref[...] + y_ref[...]) + b_ref[...]
    mu = jnp.mean(z, axis=0, keepdims=True)
    zc = z - mu
    var = jnp.mean(zc * zc, axis=0, keepdims=True)
    h = jnp.maximum(zc * lax.rsqrt(var + 1e-5) * g_ref[...] + be_ref[...], 0.0)
    y2_ref[...] = dinv * jnp.dot(h, w2_ref[...], preferred_element_type=F32)


def _tc_post2(acc0_ref, acc1_ref, y_ref, dinv_ref, b_ref, g_ref, be_ref,
              h2_ref, tab_ref):
    dinv = dinv_ref[...]
    z = dinv * (acc0_ref[...] + acc1_ref[...] + y_ref[...]) + b_ref[...]
    mu = jnp.mean(z, axis=0, keepdims=True)
    zc = z - mu
    var = jnp.mean(zc * zc, axis=0, keepdims=True)
    h2 = jnp.maximum(zc * lax.rsqrt(var + 1e-5) * g_ref[...] + be_ref[...],
                     0.0)
    h2_ref[...] = h2
    t = h2 + 1e-7
    m = jnp.max(t, axis=0, keepdims=True)
    a = jnp.exp(t - m)
    tab_ref[0:N, :] = a
    tab_ref[N:2 * N, :] = a * t


def _tc_k4(h2_ref, den_ref, num_ref, gw1_ref, gb1_ref, gw2_ref, gb2_ref,
           wo_ref, bo_ref, batch_ref, out_ref):
    aggr = num_ref[...] / (den_ref[...] + 1e-30)
    h3 = h2_ref[...] + aggr
    r = jnp.maximum(
        jnp.dot(h3, gw1_ref[...], preferred_element_type=F32) + gb1_ref[...],
        0.0)
    h4 = jnp.dot(r, gw2_ref[...], preferred_element_type=F32) + gb2_ref[...]
    gids = lax.broadcasted_iota(jnp.int32, (NG, N), 0)
    oh = (gids == batch_ref[...]).astype(F32)
    sums = jnp.dot(oh, h4, preferred_element_type=F32)
    cnt = jnp.sum(oh, axis=1, keepdims=True)
    pooled = sums / jnp.maximum(cnt, 1.0)
    out_ref[...] = (jnp.dot(pooled, wo_ref[...], preferred_element_type=F32)
                    + bo_ref[...])


def _pc(body, out_shapes):
    return pl.pallas_call(body, out_shape=out_shapes)


def kernel(x, edge_index, batch, atom_emb, W1, b1, gamma1, beta1, W2, b2,
           gamma2, beta2, gen_W1, gen_b1, gen_W2, gen_b2, W_out, b_out):
    x = x.astype(jnp.int32)
    src = edge_index[0].astype(jnp.int32)
    dst = edge_index[1].astype(jnp.int32)
    batch = batch.astype(jnp.int32)

    xi = x + 119 * jnp.arange(9, dtype=jnp.int32)[None, :]
    xi = jnp.pad(xi, ((0, NPAD - N), (0, 0)))
    idx2d = (xi.reshape(NPAD // 80, 80, 9)
             .transpose(0, 2, 1))
    idx2d = jnp.pad(idx2d, ((0, 0), (0, 7), (0, 0)))
    idx2d = idx2d.reshape(NPAD // 80 * 16, 80)
    emb_tab = atom_emb.reshape(9 * 119, H)

    pad_ar = jnp.arange(EPAD - E, dtype=jnp.int32)
    src_p = jnp.concatenate([src, (pad_ar * 131) % N])
    dst_p = jnp.concatenate([dst, N + (pad_ar % (NPAD - N))])
    src2d = src_p.reshape(ER, C)
    dst2d = dst_p.reshape(ER, C)
    srcg = jnp.concatenate([src_p, src_p + N]).reshape(2 * ER, C)

    dstdeg2d = jnp.concatenate(
        [dst, NPAD + jnp.arange(EPAD - E, dtype=jnp.int32)]).reshape(ER, C)
    h0p, degp = _make_sc_enc_deg()(idx2d, dstdeg2d, emb_tab)
    h0 = h0p[:N]
    degp = degp.reshape(NC, NPAD)
    deg = 1.0 + degp[0, :N] + degp[1, :N]

    y1, dinv = _pc(_tc_k1, [jax.ShapeDtypeStruct((N, H), F32),
                            jax.ShapeDtypeStruct((N, 1), F32)])(
        h0, deg.reshape(N, 1), W1)

    edge_pass_split = _make_sc_edge_pass(True)
    edge_pass_full = _make_sc_edge_pass(False)

    (acc1,) = edge_pass_split(y1, src2d, dst2d)
    (y2,) = _pc(_tc_post, [jax.ShapeDtypeStruct((N, H), F32)])(
        acc1[0, :N], acc1[1, :N], y1, dinv, b1.reshape(1, H),
        gamma1.reshape(1, H), beta1.reshape(1, H), W2)

    (acc2,) = edge_pass_split(y2, src2d, dst2d)
    h2, tab = _pc(_tc_post2, [jax.ShapeDtypeStruct((N, H), F32),
                              jax.ShapeDtypeStruct((2 * N, H), F32)])(
        acc2[0, :N], acc2[1, :N], y2, dinv, b2.reshape(1, H),
        gamma2.reshape(1, H), beta2.reshape(1, H))

    (gen,) = edge_pass_full(tab, srcg, dst2d)
    den, num = gen[0, :N], gen[1, :N]

    (out,) = _pc(_tc_k4, [jax.ShapeDtypeStruct((NG, H), F32)])(
        h2, den, num, gen_W1, gen_b1.reshape(1, 2 * H), gen_W2,
        gen_b2.reshape(1, H), W_out, b_out.reshape(1, H),
        batch.reshape(1, N))
    return out


# speedup vs baseline: 19.6988x; 1.0001x over previous
import functools

import jax
import jax.numpy as jnp
from jax import lax
from jax.experimental import pallas as pl
from jax.experimental.pallas import tpu as pltpu
from jax.experimental.pallas import tpu_sc as plsc

N = 10000
E = 320000
H = 128
NG = 256
NPAD = 10240
EPAD = 327680
ER = EPAD // 128
C = 128
RB = 16
NC = 2
NS = 16
DEGPAD = NPAD + (EPAD - E)

F32 = jnp.float32


def _mesh():
    return plsc.VectorSubcoreMesh(core_axis_name="c", subcore_axis_name="s",
                                  num_cores=NC, num_subcores=NS)


def _sc_enc_deg_body(idx2d, dst2d, emb, h0p, degout, ibuf, rbuf, rbuf2, abuf,
                     dbuf, ones_b, zbufd, deg_sh, sem, sem2):
    c = lax.axis_index("c")
    s = lax.axis_index("s")
    w = s * NC + c

    @pl.loop(0, 40)
    def _(i):
        zbufd[pl.ds(i * 16, 16)] = jnp.zeros((16,), F32)

    @pl.loop(0, 8)
    def _(i):
        ones_b[pl.ds(i * 16, 16)] = jnp.ones((16,), F32)

    pltpu.sync_copy(zbufd, deg_sh.at[pl.ds(s * 640, 640)])

    @pl.loop(0, 4)
    def _(k):
        chunk = w * 4 + k
        pltpu.sync_copy(idx2d.at[pl.ds(chunk * 16, 16)], ibuf)
        rbufs = (rbuf, rbuf2)
        gsems = (sem, sem2)
        gd = [None] * 9
        gd[0] = pltpu.async_copy(emb.at[ibuf.at[0]], rbufs[0], gsems[0])
        for col in range(9):
            if col + 1 < 9:
                gd[col + 1] = pltpu.async_copy(emb.at[ibuf.at[col + 1]],
                                               rbufs[(col + 1) % 2],
                                               gsems[(col + 1) % 2])
            gd[col].wait()
            rb = rbufs[col % 2]
            if col == 0:
                @pl.loop(0, 80)
                def _(r):
                    for v in range(8):
                        abuf[r, pl.ds(v * 16, 16)] = rb[r, pl.ds(v * 16, 16)]
            else:
                @pl.loop(0, 80)
                def _(r):
                    for v in range(8):
                        abuf[r, pl.ds(v * 16, 16)] = (
                            abuf[r, pl.ds(v * 16, 16)]
                            + rb[r, pl.ds(v * 16, 16)])
        pltpu.sync_copy(abuf, h0p.at[pl.ds(chunk * 80, 80)])

    plsc.subcore_barrier()

    @pl.loop(0, 5)
    def _(b):
        rowbase = ((s * 5 + b) * NC + c) * RB
        pltpu.sync_copy(dst2d.at[pl.ds(rowbase, RB)], dbuf)

        sd = [None] * RB
        for j in range(RB):
            sd[j] = pltpu.async_copy(ones_b, deg_sh.at[dbuf.at[j]], sem,
                                     add=True)
        for j in range(RB):
            sd[j].wait()

    plsc.subcore_barrier()
    pltpu.sync_copy(deg_sh.at[pl.ds(s * 640, 640)],
                    degout.at[pl.ds(c * NPAD + s * 640, 640)])


def _make_sc_enc_deg():
    return functools.partial(
        pl.kernel,
        out_type=[
            jax.ShapeDtypeStruct((NPAD, H), F32),
            jax.ShapeDtypeStruct((NC * NPAD,), F32),
        ],
        mesh=_mesh(),
        scratch_types=[
            pltpu.VMEM((16, 80), jnp.int32),
            pltpu.VMEM((80, H), F32),
            pltpu.VMEM((80, H), F32),
            pltpu.VMEM((80, H), F32),
            pltpu.VMEM((RB, C), jnp.int32),
            pltpu.VMEM((C,), F32),
            pltpu.VMEM((640,), F32),
            pltpu.VMEM_SHARED((DEGPAD,), F32),
            pltpu.SemaphoreType.DMA,
            pltpu.SemaphoreType.DMA,
        ],
    )(_sc_enc_deg_body)


def _make_sc_edge_pass(split):
    nblocks = 5 if split else 10
    NSLOT = 2

    def body(table, s2d, d2d, accout, sbuf, dbuf, rb0, rb1, acc_sh,
             gs0, gs1, ss0, ss1):
        rbufs = (rb0, rb1)
        gsems = (gs0, gs1)
        ssems = (ss0, ss1)
        cc = lax.axis_index("c")
        s = lax.axis_index("s")

        @pl.loop(0, 128)
        def _(r):
            for v in range(8):
                rb0[r, pl.ds(v * 16, 16)] = jnp.zeros((16,), F32)

        for k in range(5):
            pltpu.sync_copy(rb0, acc_sh.at[pl.ds(s * 640 + k * 128, 128)])
        plsc.subcore_barrier()

        @pl.loop(0, nblocks)
        def _(b):
            if split:
                srow = cc * (ER // 2) + s * 80 + b * RB
                drow = srow
            else:
                srow = cc * ER + s * 160 + b * RB
                drow = s * 160 + b * RB
            pltpu.sync_copy(s2d.at[pl.ds(srow, RB)], sbuf)
            pltpu.sync_copy(d2d.at[pl.ds(drow, RB)], dbuf)

            gd = [None] * RB
            sd = [None] * RB
            for j in range(RB):
                slot = j % NSLOT
                if j >= NSLOT:
                    sd[j - NSLOT].wait()
                gd[j] = pltpu.async_copy(table.at[sbuf.at[j]], rbufs[slot],
                                         gsems[slot])
                if j >= 1:
                    k = j - 1
                    gd[k].wait()
                    sd[k] = pltpu.async_copy(rbufs[k % NSLOT],
                                             acc_sh.at[dbuf.at[k]],
                                             ssems[k % NSLOT], add=True)
            gd[RB - 1].wait()
            sd[RB - 1] = pltpu.async_copy(rbufs[(RB - 1) % NSLOT],
                                          acc_sh.at[dbuf.at[RB - 1]],
                                          ssems[(RB - 1) % NSLOT], add=True)
            sd[RB - 2].wait()
            sd[RB - 1].wait()

        plsc.subcore_barrier()
        for k in range(5):
            pltpu.sync_copy(acc_sh.at[pl.ds(s * 640 + k * 128, 128)],
                            accout.at[cc, pl.ds(s * 640 + k * 128, 128)])

    return functools.partial(
        pl.kernel,
        out_type=[jax.ShapeDtypeStruct((NC, NPAD, H), F32)],
        mesh=_mesh(),
        scratch_types=[
            pltpu.VMEM((RB, C), jnp.int32),
            pltpu.VMEM((RB, C), jnp.int32),
            pltpu.VMEM((C, H), F32),
            pltpu.VMEM((C, H), F32),
            pltpu.VMEM_SHARED((NPAD, H), F32),
            pltpu.SemaphoreType.DMA,
            pltpu.SemaphoreType.DMA,
            pltpu.SemaphoreType.DMA,
            pltpu.SemaphoreType.DMA,
        ],
    )(body)


def _tc_k1(h0_ref, deg_ref, w1_ref, y1_ref, dinv_ref):
    deg = deg_ref[...]
    dinv = lax.rsqrt(jnp.maximum(deg, 1.0))
    dinv_ref[...] = dinv
    xw = jnp.dot(h0_ref[...], w1_ref[...], preferred_element_type=F32)
    y1_ref[...] = dinv * xw


def _tc_post(acc0_ref, acc1_ref, y_ref, dinv_ref, b_ref, g_ref, be_ref,
             w2_ref, y2_ref):
    dinv = dinv_ref[...]
    z = dinv * (acc0_ref[...] + acc1_ref[...] + y_ref[...]) + b_ref[...]
    mu = jnp.mean(z, axis=0, keepdims=True)
    zc = z - mu
    var = jnp.mean(zc * zc, axis=0, keepdims=True)
    h = jnp.maximum(zc * lax.rsqrt(var + 1e-5) * g_ref[...] + be_ref[...], 0.0)
    y2_ref[...] = dinv * jnp.dot(h, w2_ref[...], preferred_element_type=F32)


def _tc_post2(acc0_ref, acc1_ref, y_ref, dinv_ref, b_ref, g_ref, be_ref,
              h2_ref, tab_ref):
    dinv = dinv_ref[...]
    z = dinv * (acc0_ref[...] + acc1_ref[...] + y_ref[...]) + b_ref[...]
    mu = jnp.mean(z, axis=0, keepdims=True)
    zc = z - mu
    var = jnp.mean(zc * zc, axis=0, keepdims=True)
    h2 = jnp.maximum(zc * lax.rsqrt(var + 1e-5) * g_ref[...] + be_ref[...],
                     0.0)
    h2_ref[...] = h2
    t = h2 + 1e-7
    m = jnp.max(t, axis=0, keepdims=True)
    a = jnp.exp(t - m)
    tab_ref[0:N, :] = a
    tab_ref[N:2 * N, :] = a * t


def _tc_k4(h2_ref, den_ref, num_ref, gw1_ref, gb1_ref, gw2_ref, gb2_ref,
           wo_ref, bo_ref, batch_ref, out_ref):
    aggr = num_ref[...] / (den_ref[...] + 1e-30)
    h3 = h2_ref[...] + aggr
    r = jnp.maximum(
        jnp.dot(h3, gw1_ref[...], preferred_element_type=F32) + gb1_ref[...],
        0.0)
    h4 = jnp.dot(r, gw2_ref[...], preferred_element_type=F32) + gb2_ref[...]
    gids = lax.broadcasted_iota(jnp.int32, (NG, N), 0)
    oh = (gids == batch_ref[...]).astype(F32)
    sums = jnp.dot(oh, h4, preferred_element_type=F32)
    cnt = jnp.sum(oh, axis=1, keepdims=True)
    pooled = sums / jnp.maximum(cnt, 1.0)
    out_ref[...] = (jnp.dot(pooled, wo_ref[...], preferred_element_type=F32)
                    + bo_ref[...])


def _pc(body, out_shapes):
    return pl.pallas_call(body, out_shape=out_shapes)


def kernel(x, edge_index, batch, atom_emb, W1, b1, gamma1, beta1, W2, b2,
           gamma2, beta2, gen_W1, gen_b1, gen_W2, gen_b2, W_out, b_out):
    x = x.astype(jnp.int32)
    src = edge_index[0].astype(jnp.int32)
    dst = edge_index[1].astype(jnp.int32)
    batch = batch.astype(jnp.int32)

    xi = x + 119 * jnp.arange(9, dtype=jnp.int32)[None, :]
    xi = jnp.pad(xi, ((0, NPAD - N), (0, 0)))
    idx2d = (xi.reshape(NPAD // 80, 80, 9)
             .transpose(0, 2, 1))
    idx2d = jnp.pad(idx2d, ((0, 0), (0, 7), (0, 0)))
    idx2d = idx2d.reshape(NPAD // 80 * 16, 80)
    emb_tab = atom_emb.reshape(9 * 119, H)

    pad_ar = jnp.arange(EPAD - E, dtype=jnp.int32)
    src_p = jnp.concatenate([src, (pad_ar * 131) % N])
    dst_p = jnp.concatenate([dst, N + (pad_ar % (NPAD - N))])
    src2d = src_p.reshape(ER, C)
    dst2d = dst_p.reshape(ER, C)
    srcg = jnp.concatenate([src_p, src_p + N]).reshape(2 * ER, C)

    dstdeg2d = jnp.concatenate(
        [dst, NPAD + jnp.arange(EPAD - E, dtype=jnp.int32)]).reshape(ER, C)
    h0p, degp = _make_sc_enc_deg()(idx2d, dstdeg2d, emb_tab)
    h0 = h0p[:N]
    degp = degp.reshape(NC, NPAD)
    deg = 1.0 + degp[0, :N] + degp[1, :N]

    y1, dinv = _pc(_tc_k1, [jax.ShapeDtypeStruct((N, H), F32),
                            jax.ShapeDtypeStruct((N, 1), F32)])(
        h0, deg.reshape(N, 1), W1)

    edge_pass_split = _make_sc_edge_pass(True)
    edge_pass_full = _make_sc_edge_pass(False)

    (acc1,) = edge_pass_split(y1, src2d, dst2d)
    (y2,) = _pc(_tc_post, [jax.ShapeDtypeStruct((N, H), F32)])(
        acc1[0, :N], acc1[1, :N], y1, dinv, b1.reshape(1, H),
        gamma1.reshape(1, H), beta1.reshape(1, H), W2)

    (acc2,) = edge_pass_split(y2, src2d, dst2d)
    h2, tab = _pc(_tc_post2, [jax.ShapeDtypeStruct((N, H), F32),
                              jax.ShapeDtypeStruct((2 * N, H), F32)])(
        acc2[0, :N], acc2[1, :N], y2, dinv, b2.reshape(1, H),
        gamma2.reshape(1, H), beta2.reshape(1, H))

    (gen,) = edge_pass_full(tab, srcg, dst2d)
    den, num = gen[0, :N], gen[1, :N]

    (out,) = _pc(_tc_k4, [jax.ShapeDtypeStruct((NG, H), F32)])(
        h2, den, num, gen_W1, gen_b1.reshape(1, 2 * H), gen_W2,
        gen_b2.reshape(1, H), W_out, b_out.reshape(1, H),
        batch.reshape(1, N))
    return out


# speedup vs baseline: 19.7256x; 1.0014x over previous
import functools

import jax
import jax.numpy as jnp
from jax import lax
from jax.experimental import pallas as pl
from jax.experimental.pallas import tpu as pltpu
from jax.experimental.pallas import tpu_sc as plsc

N = 10000
E = 320000
H = 128
NG = 256
NPAD = 10240
EPAD = 327680
ER = EPAD // 128
C = 128
RB = 16
NC = 2
NS = 16
DEGPAD = NPAD + (EPAD - E)

F32 = jnp.float32


def _mesh():
    return plsc.VectorSubcoreMesh(core_axis_name="c", subcore_axis_name="s",
                                  num_cores=NC, num_subcores=NS)


def _sc_enc_deg_body(idx2d, dst2d, emb, h0p, degout, ibuf, rbuf, rbuf2, abuf,
                     dbuf, ones_b, zbufd, deg_sh, sem, sem2):
    c = lax.axis_index("c")
    s = lax.axis_index("s")
    w = c * NS + s

    @pl.loop(0, 40)
    def _(i):
        zbufd[pl.ds(i * 16, 16)] = jnp.zeros((16,), F32)

    @pl.loop(0, 8)
    def _(i):
        ones_b[pl.ds(i * 16, 16)] = jnp.ones((16,), F32)

    pltpu.sync_copy(zbufd, deg_sh.at[pl.ds(s * 640, 640)])

    @pl.loop(0, 4)
    def _(k):
        chunk = w * 4 + k
        pltpu.sync_copy(idx2d.at[pl.ds(chunk * 16, 16)], ibuf)
        rbufs = (rbuf, rbuf2)
        gsems = (sem, sem2)
        gd = [None] * 9
        gd[0] = pltpu.async_copy(emb.at[ibuf.at[0]], rbufs[0], gsems[0])
        for col in range(9):
            if col + 1 < 9:
                gd[col + 1] = pltpu.async_copy(emb.at[ibuf.at[col + 1]],
                                               rbufs[(col + 1) % 2],
                                               gsems[(col + 1) % 2])
            gd[col].wait()
            rb = rbufs[col % 2]
            if col == 0:
                @pl.loop(0, 80)
                def _(r):
                    for v in range(8):
                        abuf[r, pl.ds(v * 16, 16)] = rb[r, pl.ds(v * 16, 16)]
            else:
                @pl.loop(0, 80)
                def _(r):
                    for v in range(8):
                        abuf[r, pl.ds(v * 16, 16)] = (
                            abuf[r, pl.ds(v * 16, 16)]
                            + rb[r, pl.ds(v * 16, 16)])
        pltpu.sync_copy(abuf, h0p.at[pl.ds(chunk * 80, 80)])

    plsc.subcore_barrier()

    @pl.loop(0, 5)
    def _(b):
        rowbase = ((s * 5 + b) * NC + c) * RB
        pltpu.sync_copy(dst2d.at[pl.ds(rowbase, RB)], dbuf)

        sd = [None] * RB
        for j in range(RB):
            sd[j] = pltpu.async_copy(ones_b, deg_sh.at[dbuf.at[j]], sem,
                                     add=True)
        for j in range(RB):
            sd[j].wait()

    plsc.subcore_barrier()
    pltpu.sync_copy(deg_sh.at[pl.ds(s * 640, 640)],
                    degout.at[pl.ds(c * NPAD + s * 640, 640)])


def _make_sc_enc_deg():
    return functools.partial(
        pl.kernel,
        out_type=[
            jax.ShapeDtypeStruct((NPAD, H), F32),
            jax.ShapeDtypeStruct((NC * NPAD,), F32),
        ],
        mesh=_mesh(),
        scratch_types=[
            pltpu.VMEM((16, 80), jnp.int32),
            pltpu.VMEM((80, H), F32),
            pltpu.VMEM((80, H), F32),
            pltpu.VMEM((80, H), F32),
            pltpu.VMEM((RB, C), jnp.int32),
            pltpu.VMEM((C,), F32),
            pltpu.VMEM((640,), F32),
            pltpu.VMEM_SHARED((DEGPAD,), F32),
            pltpu.SemaphoreType.DMA,
            pltpu.SemaphoreType.DMA,
        ],
    )(_sc_enc_deg_body)


def _make_sc_edge_pass(split):
    nblocks = 5 if split else 10
    NSLOT = 2

    def body(table, s2d, d2d, accout, sbuf, dbuf, rb0, rb1, acc_sh,
             gs0, gs1, ss0, ss1):
        rbufs = (rb0, rb1)
        gsems = (gs0, gs1)
        ssems = (ss0, ss1)
        cc = lax.axis_index("c")
        s = lax.axis_index("s")

        @pl.loop(0, 128)
        def _(r):
            for v in range(8):
                rb0[r, pl.ds(v * 16, 16)] = jnp.zeros((16,), F32)

        for k in range(5):
            pltpu.sync_copy(rb0, acc_sh.at[pl.ds(s * 640 + k * 128, 128)])
        plsc.subcore_barrier()

        @pl.loop(0, nblocks)
        def _(b):
            if split:
                srow = cc * (ER // 2) + s * 80 + b * RB
                drow = srow
            else:
                srow = cc * ER + s * 160 + b * RB
                drow = s * 160 + b * RB
            pltpu.sync_copy(s2d.at[pl.ds(srow, RB)], sbuf)
            pltpu.sync_copy(d2d.at[pl.ds(drow, RB)], dbuf)

            gd = [None] * RB
            sd = [None] * RB
            for j in range(RB):
                slot = j % NSLOT
                if j >= NSLOT:
                    sd[j - NSLOT].wait()
                gd[j] = pltpu.async_copy(table.at[sbuf.at[j]], rbufs[slot],
                                         gsems[slot])
                if j >= 1:
                    k = j - 1
                    gd[k].wait()
                    sd[k] = pltpu.async_copy(rbufs[k % NSLOT],
                                             acc_sh.at[dbuf.at[k]],
                                             ssems[k % NSLOT], add=True)
            gd[RB - 1].wait()
            sd[RB - 1] = pltpu.async_copy(rbufs[(RB - 1) % NSLOT],
                                          acc_sh.at[dbuf.at[RB - 1]],
                                          ssems[(RB - 1) % NSLOT], add=True)
            sd[RB - 2].wait()
            sd[RB - 1].wait()

        plsc.subcore_barrier()
        for k in range(5):
            pltpu.sync_copy(acc_sh.at[pl.ds(s * 640 + k * 128, 128)],
                            accout.at[cc, pl.ds(s * 640 + k * 128, 128)])

    return functools.partial(
        pl.kernel,
        out_type=[jax.ShapeDtypeStruct((NC, NPAD, H), F32)],
        mesh=_mesh(),
        scratch_types=[
            pltpu.VMEM((RB, C), jnp.int32),
            pltpu.VMEM((RB, C), jnp.int32),
            pltpu.VMEM((C, H), F32),
            pltpu.VMEM((C, H), F32),
            pltpu.VMEM_SHARED((NPAD, H), F32),
            pltpu.SemaphoreType.DMA,
            pltpu.SemaphoreType.DMA,
            pltpu.SemaphoreType.DMA,
            pltpu.SemaphoreType.DMA,
        ],
    )(body)


def _tc_k1(h0_ref, deg_ref, w1_ref, y1_ref, dinv_ref):
    deg = deg_ref[...]
    dinv = lax.rsqrt(jnp.maximum(deg, 1.0))
    dinv_ref[...] = dinv
    xw = jnp.dot(h0_ref[...], w1_ref[...], preferred_element_type=F32)
    y1_ref[...] = dinv * xw


def _tc_post(acc0_ref, acc1_ref, y_ref, dinv_ref, b_ref, g_ref, be_ref,
             w2_ref, y2_ref):
    dinv = dinv_ref[...]
    z = dinv * (acc0_ref[...] + acc1_ref[...] + y_ref[...]) + b_ref[...]
    mu = jnp.mean(z, axis=0, keepdims=True)
    zc = z - mu
    var = jnp.mean(zc * zc, axis=0, keepdims=True)
    h = jnp.maximum(zc * lax.rsqrt(var + 1e-5) * g_ref[...] + be_ref[...], 0.0)
    y2_ref[...] = dinv * jnp.dot(h, w2_ref[...], preferred_element_type=F32)


def _tc_post2(acc0_ref, acc1_ref, y_ref, dinv_ref, b_ref, g_ref, be_ref,
              h2_ref, tab_ref):
    dinv = dinv_ref[...]
    z = dinv * (acc0_ref[...] + acc1_ref[...] + y_ref[...]) + b_ref[...]
    mu = jnp.mean(z, axis=0, keepdims=True)
    zc = z - mu
    var = jnp.mean(zc * zc, axis=0, keepdims=True)
    h2 = jnp.maximum(zc * lax.rsqrt(var + 1e-5) * g_ref[...] + be_ref[...],
                     0.0)
    h2_ref[...] = h2
    t = h2 + 1e-7
    m = jnp.max(t, axis=0, keepdims=True)
    a = jnp.exp(t - m)
    tab_ref[0:N, :] = a
    tab_ref[N:2 * N, :] = a * t


def _tc_k4(h2_ref, den_ref, num_ref, gw1_ref, gb1_ref, gw2_ref, gb2_ref,
           wo_ref, bo_ref, batch_ref, out_ref):
    aggr = num_ref[...] / (den_ref[...] + 1e-30)
    h3 = h2_ref[...] + aggr
    r = jnp.maximum(
        jnp.dot(h3, gw1_ref[...], preferred_element_type=F32) + gb1_ref[...],
        0.0)
    h4 = jnp.dot(r, gw2_ref[...], preferred_element_type=F32) + gb2_ref[...]
    gids = lax.broadcasted_iota(jnp.int32, (NG, N), 0)
    oh = (gids == batch_ref[...]).astype(F32)
    sums = jnp.dot(oh, h4, preferred_element_type=F32)
    cnt = jnp.sum(oh, axis=1, keepdims=True)
    pooled = sums / jnp.maximum(cnt, 1.0)
    out_ref[...] = (jnp.dot(pooled, wo_ref[...], preferred_element_type=F32)
                    + bo_ref[...])


def _pc(body, out_shapes):
    return pl.pallas_call(body, out_shape=out_shapes)


def kernel(x, edge_index, batch, atom_emb, W1, b1, gamma1, beta1, W2, b2,
           gamma2, beta2, gen_W1, gen_b1, gen_W2, gen_b2, W_out, b_out):
    x = x.astype(jnp.int32)
    src = edge_index[0].astype(jnp.int32)
    dst = edge_index[1].astype(jnp.int32)
    batch = batch.astype(jnp.int32)

    xi = x + 119 * jnp.arange(9, dtype=jnp.int32)[None, :]
    xi = jnp.pad(xi, ((0, NPAD - N), (0, 0)))
    idx2d = (xi.reshape(NPAD // 80, 80, 9)
             .transpose(0, 2, 1))
    idx2d = jnp.pad(idx2d, ((0, 0), (0, 7), (0, 0)))
    idx2d = idx2d.reshape(NPAD // 80 * 16, 80)
    emb_tab = atom_emb.reshape(9 * 119, H)

    pad_ar = jnp.arange(EPAD - E, dtype=jnp.int32)
    src_p = jnp.concatenate([src, (pad_ar * 131) % N])
    dst_p = jnp.concatenate([dst, N + (pad_ar % (NPAD - N))])
    src2d = src_p.reshape(ER, C)
    dst2d = dst_p.reshape(ER, C)
    srcg = jnp.concatenate([src_p, src_p + N]).reshape(2 * ER, C)

    dstdeg2d = jnp.concatenate(
        [dst, NPAD + jnp.arange(EPAD - E, dtype=jnp.int32)]).reshape(ER, C)
    h0p, degp = _make_sc_enc_deg()(idx2d, dstdeg2d, emb_tab)
    h0 = h0p[:N]
    degp = degp.reshape(NC, NPAD)
    deg = 1.0 + degp[0, :N] + degp[1, :N]

    y1, dinv = _pc(_tc_k1, [jax.ShapeDtypeStruct((N, H), F32),
                            jax.ShapeDtypeStruct((N, 1), F32)])(
        h0, deg.reshape(N, 1), W1)

    edge_pass_split = _make_sc_edge_pass(True)
    edge_pass_full = _make_sc_edge_pass(False)

    (acc1,) = edge_pass_split(y1, src2d, dst2d)
    (y2,) = _pc(_tc_post, [jax.ShapeDtypeStruct((N, H), F32)])(
        acc1[0, :N], acc1[1, :N], y1, dinv, b1.reshape(1, H),
        gamma1.reshape(1, H), beta1.reshape(1, H), W2)

    (acc2,) = edge_pass_split(y2, src2d, dst2d)
    h2, tab = _pc(_tc_post2, [jax.ShapeDtypeStruct((N, H), F32),
                              jax.ShapeDtypeStruct((2 * N, H), F32)])(
        acc2[0, :N], acc2[1, :N], y2, dinv, b2.reshape(1, H),
        gamma2.reshape(1, H), beta2.reshape(1, H))

    (gen,) = edge_pass_full(tab, srcg, dst2d)
    den, num = gen[0, :N], gen[1, :N]

    (out,) = _pc(_tc_k4, [jax.ShapeDtypeStruct((NG, H), F32)])(
        h2, den, num, gen_W1, gen_b1.reshape(1, 2 * H), gen_W2,
        gen_b2.reshape(1, H), W_out, b_out.reshape(1, H),
        batch.reshape(1, N))
    return out


# speedup vs baseline: 19.7668x; 1.0021x over previous
import functools

import jax
import jax.numpy as jnp
from jax import lax
from jax.experimental import pallas as pl
from jax.experimental.pallas import tpu as pltpu
from jax.experimental.pallas import tpu_sc as plsc

N = 10000
E = 320000
H = 128
NG = 256
NPAD = 10240
EPAD = 327680
ER = EPAD // 128
C = 128
RB = 16
NC = 2
NS = 16
DEGPAD = NPAD + (EPAD - E)

F32 = jnp.float32


def _mesh():
    return plsc.VectorSubcoreMesh(core_axis_name="c", subcore_axis_name="s",
                                  num_cores=NC, num_subcores=NS)


def _sc_enc_deg_body(idx2d, dst2d, emb, h0p, degout, ibuf, rbuf, rbuf2, abuf,
                     dbuf, ones_b, zbufd, nidx, deg_sh, hacc_sh, sem, sem2,
                     ssem):
    c = lax.axis_index("c")
    s = lax.axis_index("s")
    w = c * NS + s

    @pl.loop(0, 40)
    def _(i):
        zbufd[pl.ds(i * 16, 16)] = jnp.zeros((16,), F32)

    @pl.loop(0, 8)
    def _(i):
        ones_b[pl.ds(i * 16, 16)] = jnp.ones((16,), F32)

    pltpu.sync_copy(zbufd, deg_sh.at[pl.ds(s * 640, 640)])

    @pl.loop(0, 80)
    def _(r):
        for v in range(8):
            abuf[r, pl.ds(v * 16, 16)] = jnp.zeros((16,), F32)

    for k in range(4):
        pltpu.sync_copy(abuf, hacc_sh.at[pl.ds(w * 320 + k * 80, 80)])

    @pl.loop(0, 4)
    def _(k):
        chunk = w * 4 + k
        base = chunk * 80

        @pl.loop(0, 5)
        def _(v):
            nidx[pl.ds(v * 16, 16)] = (
                lax.broadcasted_iota(jnp.int32, (16,), 0) + base + v * 16)

        pltpu.sync_copy(idx2d.at[pl.ds(chunk * 16, 16)], ibuf)
        rbufs = (rbuf, rbuf2)
        gsems = (sem, sem2)
        gd = [None] * 9
        sd = [None] * 9
        gd[0] = pltpu.async_copy(emb.at[ibuf.at[0]], rbufs[0], gsems[0])
        for col in range(9):
            if col + 1 < 9:
                gd[col + 1] = pltpu.async_copy(emb.at[ibuf.at[col + 1]],
                                               rbufs[(col + 1) % 2],
                                               gsems[(col + 1) % 2])
            gd[col].wait()
            if col >= 2:
                sd[col - 2].wait()
            sd[col] = pltpu.async_copy(rbufs[col % 2], hacc_sh.at[nidx],
                                       ssem, add=True)
        sd[7].wait()
        sd[8].wait()
        pltpu.sync_copy(hacc_sh.at[pl.ds(base, 80)],
                        h0p.at[pl.ds(base, 80)])

    plsc.subcore_barrier()

    @pl.loop(0, 5)
    def _(b):
        rowbase = ((s * 5 + b) * NC + c) * RB
        pltpu.sync_copy(dst2d.at[pl.ds(rowbase, RB)], dbuf)

        sd = [None] * RB
        for j in range(RB):
            sd[j] = pltpu.async_copy(ones_b, deg_sh.at[dbuf.at[j]], sem,
                                     add=True)
        for j in range(RB):
            sd[j].wait()

    plsc.subcore_barrier()
    pltpu.sync_copy(deg_sh.at[pl.ds(s * 640, 640)],
                    degout.at[pl.ds(c * NPAD + s * 640, 640)])


def _make_sc_enc_deg():
    return functools.partial(
        pl.kernel,
        out_type=[
            jax.ShapeDtypeStruct((NPAD, H), F32),
            jax.ShapeDtypeStruct((NC * NPAD,), F32),
        ],
        mesh=_mesh(),
        scratch_types=[
            pltpu.VMEM((16, 80), jnp.int32),
            pltpu.VMEM((80, H), F32),
            pltpu.VMEM((80, H), F32),
            pltpu.VMEM((80, H), F32),
            pltpu.VMEM((RB, C), jnp.int32),
            pltpu.VMEM((C,), F32),
            pltpu.VMEM((640,), F32),
            pltpu.VMEM((80,), jnp.int32),
            pltpu.VMEM_SHARED((DEGPAD,), F32),
            pltpu.VMEM_SHARED((NPAD, H), F32),
            pltpu.SemaphoreType.DMA,
            pltpu.SemaphoreType.DMA,
            pltpu.SemaphoreType.DMA,
        ],
    )(_sc_enc_deg_body)


def _make_sc_edge_pass(split):
    nblocks = 5 if split else 10
    NSLOT = 2

    def body(table, s2d, d2d, accout, sbuf, dbuf, rb0, rb1, acc_sh,
             gs0, gs1, ss0, ss1):
        rbufs = (rb0, rb1)
        gsems = (gs0, gs1)
        ssems = (ss0, ss1)
        cc = lax.axis_index("c")
        s = lax.axis_index("s")

        @pl.loop(0, 128)
        def _(r):
            for v in range(8):
                rb0[r, pl.ds(v * 16, 16)] = jnp.zeros((16,), F32)

        for k in range(5):
            pltpu.sync_copy(rb0, acc_sh.at[pl.ds(s * 640 + k * 128, 128)])
        plsc.subcore_barrier()

        @pl.loop(0, nblocks)
        def _(b):
            if split:
                srow = cc * (ER // 2) + s * 80 + b * RB
                drow = srow
            else:
                srow = cc * ER + s * 160 + b * RB
                drow = s * 160 + b * RB
            pltpu.sync_copy(s2d.at[pl.ds(srow, RB)], sbuf)
            pltpu.sync_copy(d2d.at[pl.ds(drow, RB)], dbuf)

            gd = [None] * RB
            sd = [None] * RB
            for j in range(RB):
                slot = j % NSLOT
                if j >= NSLOT:
                    sd[j - NSLOT].wait()
                gd[j] = pltpu.async_copy(table.at[sbuf.at[j]], rbufs[slot],
                                         gsems[slot])
                if j >= 1:
                    k = j - 1
                    gd[k].wait()
                    sd[k] = pltpu.async_copy(rbufs[k % NSLOT],
                                             acc_sh.at[dbuf.at[k]],
                                             ssems[k % NSLOT], add=True)
            gd[RB - 1].wait()
            sd[RB - 1] = pltpu.async_copy(rbufs[(RB - 1) % NSLOT],
                                          acc_sh.at[dbuf.at[RB - 1]],
                                          ssems[(RB - 1) % NSLOT], add=True)
            sd[RB - 2].wait()
            sd[RB - 1].wait()

        plsc.subcore_barrier()
        for k in range(5):
            pltpu.sync_copy(acc_sh.at[pl.ds(s * 640 + k * 128, 128)],
                            accout.at[cc, pl.ds(s * 640 + k * 128, 128)])

    return functools.partial(
        pl.kernel,
        out_type=[jax.ShapeDtypeStruct((NC, NPAD, H), F32)],
        mesh=_mesh(),
        scratch_types=[
            pltpu.VMEM((RB, C), jnp.int32),
            pltpu.VMEM((RB, C), jnp.int32),
            pltpu.VMEM((C, H), F32),
            pltpu.VMEM((C, H), F32),
            pltpu.VMEM_SHARED((NPAD, H), F32),
            pltpu.SemaphoreType.DMA,
            pltpu.SemaphoreType.DMA,
            pltpu.SemaphoreType.DMA,
            pltpu.SemaphoreType.DMA,
        ],
    )(body)


def _tc_k1(h0_ref, deg_ref, w1_ref, y1_ref, dinv_ref):
    deg = deg_ref[...]
    dinv = lax.rsqrt(jnp.maximum(deg, 1.0))
    dinv_ref[...] = dinv
    xw = jnp.dot(h0_ref[...], w1_ref[...], preferred_element_type=F32)
    y1_ref[...] = dinv * xw


def _tc_post(acc0_ref, acc1_ref, y_ref, dinv_ref, b_ref, g_ref, be_ref,
             w2_ref, y2_ref):
    dinv = dinv_ref[...]
    z = dinv * (acc0_ref[...] + acc1_ref[...] + y_ref[...]) + b_ref[...]
    mu = jnp.mean(z, axis=0, keepdims=True)
    zc = z - mu
    var = jnp.mean(zc * zc, axis=0, keepdims=True)
    h = jnp.maximum(zc * lax.rsqrt(var + 1e-5) * g_ref[...] + be_ref[...], 0.0)
    y2_ref[...] = dinv * jnp.dot(h, w2_ref[...], preferred_element_type=F32)


def _tc_post2(acc0_ref, acc1_ref, y_ref, dinv_ref, b_ref, g_ref, be_ref,
              h2_ref, tab_ref):
    dinv = dinv_ref[...]
    z = dinv * (acc0_ref[...] + acc1_ref[...] + y_ref[...]) + b_ref[...]
    mu = jnp.mean(z, axis=0, keepdims=True)
    zc = z - mu
    var = jnp.mean(zc * zc, axis=0, keepdims=True)
    h2 = jnp.maximum(zc * lax.rsqrt(var + 1e-5) * g_ref[...] + be_ref[...],
                     0.0)
    h2_ref[...] = h2
    t = h2 + 1e-7
    m = jnp.max(t, axis=0, keepdims=True)
    a = jnp.exp(t - m)
    tab_ref[0:N, :] = a
    tab_ref[N:2 * N, :] = a * t


def _tc_k4(h2_ref, den_ref, num_ref, gw1_ref, gb1_ref, gw2_ref, gb2_ref,
           wo_ref, bo_ref, batch_ref, out_ref):
    aggr = num_ref[...] / (den_ref[...] + 1e-30)
    h3 = h2_ref[...] + aggr
    r = jnp.maximum(
        jnp.dot(h3, gw1_ref[...], preferred_element_type=F32) + gb1_ref[...],
        0.0)
    h4 = jnp.dot(r, gw2_ref[...], preferred_element_type=F32) + gb2_ref[...]
    gids = lax.broadcasted_iota(jnp.int32, (NG, N), 0)
    oh = (gids == batch_ref[...]).astype(F32)
    sums = jnp.dot(oh, h4, preferred_element_type=F32)
    cnt = jnp.sum(oh, axis=1, keepdims=True)
    pooled = sums / jnp.maximum(cnt, 1.0)
    out_ref[...] = (jnp.dot(pooled, wo_ref[...], preferred_element_type=F32)
                    + bo_ref[...])


def _pc(body, out_shapes):
    return pl.pallas_call(body, out_shape=out_shapes)


def kernel(x, edge_index, batch, atom_emb, W1, b1, gamma1, beta1, W2, b2,
           gamma2, beta2, gen_W1, gen_b1, gen_W2, gen_b2, W_out, b_out):
    x = x.astype(jnp.int32)
    src = edge_index[0].astype(jnp.int32)
    dst = edge_index[1].astype(jnp.int32)
    batch = batch.astype(jnp.int32)

    xi = x + 119 * jnp.arange(9, dtype=jnp.int32)[None, :]
    xi = jnp.pad(xi, ((0, NPAD - N), (0, 0)))
    idx2d = (xi.reshape(NPAD // 80, 80, 9)
             .transpose(0, 2, 1))
    idx2d = jnp.pad(idx2d, ((0, 0), (0, 7), (0, 0)))
    idx2d = idx2d.reshape(NPAD // 80 * 16, 80)
    emb_tab = atom_emb.reshape(9 * 119, H)

    pad_ar = jnp.arange(EPAD - E, dtype=jnp.int32)
    src_p = jnp.concatenate([src, (pad_ar * 131) % N])
    dst_p = jnp.concatenate([dst, N + (pad_ar % (NPAD - N))])
    src2d = src_p.reshape(ER, C)
    dst2d = dst_p.reshape(ER, C)
    srcg = jnp.concatenate([src_p, src_p + N]).reshape(2 * ER, C)

    dstdeg2d = jnp.concatenate(
        [dst, NPAD + jnp.arange(EPAD - E, dtype=jnp.int32)]).reshape(ER, C)
    h0p, degp = _make_sc_enc_deg()(idx2d, dstdeg2d, emb_tab)
    h0 = h0p[:N]
    degp = degp.reshape(NC, NPAD)
    deg = 1.0 + degp[0, :N] + degp[1, :N]

    y1, dinv = _pc(_tc_k1, [jax.ShapeDtypeStruct((N, H), F32),
                            jax.ShapeDtypeStruct((N, 1), F32)])(
        h0, deg.reshape(N, 1), W1)

    edge_pass_split = _make_sc_edge_pass(True)
    edge_pass_full = _make_sc_edge_pass(False)

    (acc1,) = edge_pass_split(y1, src2d, dst2d)
    (y2,) = _pc(_tc_post, [jax.ShapeDtypeStruct((N, H), F32)])(
        acc1[0, :N], acc1[1, :N], y1, dinv, b1.reshape(1, H),
        gamma1.reshape(1, H), beta1.reshape(1, H), W2)

    (acc2,) = edge_pass_split(y2, src2d, dst2d)
    h2, tab = _pc(_tc_post2, [jax.ShapeDtypeStruct((N, H), F32),
                              jax.ShapeDtypeStruct((2 * N, H), F32)])(
        acc2[0, :N], acc2[1, :N], y2, dinv, b2.reshape(1, H),
        gamma2.reshape(1, H), beta2.reshape(1, H))

    (gen,) = edge_pass_full(tab, srcg, dst2d)
    den, num = gen[0, :N], gen[1, :N]

    (out,) = _pc(_tc_k4, [jax.ShapeDtypeStruct((NG, H), F32)])(
        h2, den, num, gen_W1, gen_b1.reshape(1, 2 * H), gen_W2,
        gen_b2.reshape(1, H), W_out, b_out.reshape(1, H),
        batch.reshape(1, N))
    return out
